# baseline (speedup 1.0000x reference)
import functools
import jax
import jax.numpy as jnp
from jax import lax
from jax.experimental import pallas as pl
from jax.experimental.pallas import tpu as pltpu
from jax.experimental.pallas import tpu_sc as plsc

B, N, K, T = 64, 128, 32, 8
HID, SP, HD, HE, HL, HLO = 64, 64, 2, 32, 48, 32
L = 2
TAU = 2.0
E = N * K
NC, NS, LN = 2, 16, 16
NW = NC * NS
BPW = B // NW
NB = N // LN
DC = HID // LN


def _sc_edges_body(ajt, ait, posx, posy, idxt, maskt, wrow, accT,
                   tab_v, ai_v, px_v, py_v, idx_v, msk_v, wr_v, acc_v):
    wid = lax.axis_index("s") * NC + lax.axis_index("c")
    pltpu.sync_copy(wrow, wr_v)
    for bb in range(BPW):
        b = wid * BPW + bb
        pltpu.sync_copy(ajt.at[b], tab_v)
        pltpu.sync_copy(ait.at[b], ai_v)
        pltpu.sync_copy(posx.at[b], px_v)
        pltpu.sync_copy(posy.at[b], py_v)
        pltpu.sync_copy(idxt.at[b], idx_v)
        pltpu.sync_copy(maskt.at[b], msk_v)

        def blk_body(blk, carry_blk):
            off = blk * LN
            pxi = px_v[pl.ds(off, LN)]
            pyi = py_v[pl.ds(off, LN)]
            for c in range(DC):
                ai_c = [ai_v[c * LN + d, pl.ds(off, LN)] for d in range(LN)]
                wr_c = [wr_v[c * LN + d, :] for d in range(LN)]

                def k_body(k, carry):
                    jv = idx_v[k, pl.ds(off, LN)]
                    mv = msk_v[k, pl.ds(off, LN)]
                    gx = plsc.load_gather(px_v, [jv])
                    gy = plsc.load_gather(py_v, [jv])
                    rx = pxi - gx
                    ry = pyi - gy
                    d2 = rx * rx + ry * ry
                    out = []
                    for d in range(LN):
                        dd = c * LN + d
                        g = plsc.load_gather(
                            tab_v, [jv, jnp.full((LN,), dd, jnp.int32)])
                        v = jnp.maximum(ai_c[d] + g + d2 * wr_c[d], 0.0)
                        out.append(carry[d] + mv * v)
                    return tuple(out)

                acc0 = tuple(jnp.zeros((LN,), jnp.float32) for _ in range(LN))
                acc = lax.fori_loop(0, K, k_body, acc0)
                for d in range(LN):
                    acc_v[c * LN + d, pl.ds(off, LN)] = acc[d]
            return carry_blk

        lax.fori_loop(0, NB, blk_body, 0)
        pltpu.sync_copy(acc_v, accT.at[b])


def _sc_edges(ajt, ait, posx, posy, idxt, maskt, wrow):
    mesh = plsc.VectorSubcoreMesh(core_axis_name="c", subcore_axis_name="s")
    return pl.kernel(
        _sc_edges_body,
        mesh=mesh,
        compiler_params=pltpu.CompilerParams(needs_layout_passes=False),
        out_type=jax.ShapeDtypeStruct((B, HID, N), jnp.float32),
        scratch_types=[
            pltpu.VMEM((N, HID), jnp.float32),
            pltpu.VMEM((HID, N), jnp.float32),
            pltpu.VMEM((N,), jnp.float32),
            pltpu.VMEM((N,), jnp.float32),
            pltpu.VMEM((K, N), jnp.int32),
            pltpu.VMEM((K, N), jnp.float32),
            pltpu.VMEM((HID, LN), jnp.float32),
            pltpu.VMEM((HID, N), jnp.float32),
        ],
    )(ajt, ait, posx, posy, idxt, maskt, wrow)


def _dotT(w, h):
    return lax.dot_general(w, h, (((0,), (1,)), ((), ())),
                           preferred_element_type=jnp.float32)


def _a1_body(beta_ref, ped_ref, maskf_ref,
             Wef_ref, Wet_ref, bemb_ref, Wm1a0_ref, Wm1b0_ref, bm1c0_ref,
             h0_ref, ajt0_ref, ait0_ref, cnt_ref):
    f32 = jnp.float32
    beta = beta_ref[0, 0, 0]
    bvec = jnp.full((1, 1), beta, f32)
    sb = jnp.sin(bvec)
    cb = jnp.cos(bvec)
    trow = (beta * Wet_ref[0:1, :] + sb * Wet_ref[1:2, :]
            + cb * Wet_ref[2:3, :] + bemb_ref[...])
    ped = ped_ref[0]
    h = jnp.dot(ped, Wef_ref[...], preferred_element_type=f32) + trow
    h0_ref[0] = h
    ajt0_ref[0] = jnp.dot(h, Wm1b0_ref[...], preferred_element_type=f32)
    ait0_ref[0] = _dotT(Wm1a0_ref[...], h) + bm1c0_ref[...]
    cnt_ref[0] = jnp.sum(maskf_ref[0], axis=1, keepdims=True)


def _a2_body(beta_ref, x_ref, self_ref, hist_ref,
             Whebd_ref, bhebd_ref, WxT_ref, WhT_ref, bg_ref, Wlo_ref, blo_ref,
             Wsp_ref, bsp_ref, Wc1b_ref, Wc1c_ref, Wc1t_ref, bc1_ref,
             part_ref, pred_ref):
    f32 = jnp.float32
    beta = beta_ref[0, 0, 0]
    bvec = jnp.full((1, 1), beta, f32)
    sb = jnp.sin(bvec)
    cb = jnp.cos(bvec)

    he = jnp.maximum(
        jnp.dot(hist_ref[0], Whebd_ref[...], preferred_element_type=f32)
        + bhebd_ref[...], 0.0)
    hs = jnp.zeros((N, HL), f32)
    cs = jnp.zeros((N, HL), f32)
    for tt in range(T):
        xt = he[:, HE * tt:HE * (tt + 1)]
        gates = (jnp.dot(xt, WxT_ref[...], preferred_element_type=f32)
                 + jnp.dot(hs, WhT_ref[...], preferred_element_type=f32)
                 + bg_ref[...])
        ig = jax.nn.sigmoid(gates[:, 0:HL])
        fg = jax.nn.sigmoid(gates[:, HL:2 * HL])
        gg = jnp.tanh(gates[:, 2 * HL:3 * HL])
        og = jax.nn.sigmoid(gates[:, 3 * HL:4 * HL])
        cs = fg * cs + ig * gg
        hs = og * jnp.tanh(cs)
    hist_out = jnp.dot(hs, Wlo_ref[...], preferred_element_type=f32) + blo_ref[...]

    spatial = jnp.maximum(
        jnp.dot(x_ref[0], Wsp_ref[...], preferred_element_type=f32)
        + bsp_ref[...], 0.0)
    sf = self_ref[0]
    ds = sf[:, 4:5]
    sx = sf[:, 0:1]
    sy = sf[:, 1:2]
    temp = jnp.sqrt(sx * sx + sy * sy)
    temp_ = jnp.where(temp == 0.0, temp + 0.1, temp)
    predx = (ds * sx / temp_ - sf[:, 2:3]) / TAU
    predy = (ds * sy / temp_ - sf[:, 3:4]) / TAU
    pred_ref[0] = jnp.concatenate([predx, predy], axis=1)

    trow2 = (beta * Wc1t_ref[0:1, :] + sb * Wc1t_ref[1:2, :]
             + cb * Wc1t_ref[2:3, :] + bc1_ref[...])
    part_ref[0] = (jnp.dot(hist_out, Wc1b_ref[...], preferred_element_type=f32)
                   + jnp.dot(spatial, Wc1c_ref[...], preferred_element_type=f32)
                   + trow2)


def _layer_update(h, accT, cnt, Wm2_ref, bm2_ref, Wu1a_ref, Wu1b_ref,
                  bu1_ref, Wu2_ref, bu2_ref):
    f32 = jnp.float32
    agg = (lax.dot_general(accT, Wm2_ref[...], (((0,), (0,)), ((), ())),
                           preferred_element_type=f32)
           + cnt * bm2_ref[...])
    upd = jnp.maximum(
        jnp.dot(h, Wu1a_ref[...], preferred_element_type=f32)
        + jnp.dot(agg, Wu1b_ref[...], preferred_element_type=f32)
        + bu1_ref[...], 0.0)
    return h + jnp.dot(upd, Wu2_ref[...], preferred_element_type=f32) + bu2_ref[...]


def _b_body(h0_ref, accT0_ref, cnt_ref,
            Wm20_ref, bm20_ref, Wu1a0_ref, Wu1b0_ref, bu10_ref, Wu20_ref,
            bu20_ref, Wm1a1_ref, Wm1b1_ref, bm1c1_ref,
            h1_ref, ajt1_ref, ait1_ref):
    f32 = jnp.float32
    h1 = _layer_update(h0_ref[0], accT0_ref[0], cnt_ref[0], Wm20_ref, bm20_ref,
                       Wu1a0_ref, Wu1b0_ref, bu10_ref, Wu20_ref, bu20_ref)
    h1_ref[0] = h1
    ajt1_ref[0] = jnp.dot(h1, Wm1b1_ref[...], preferred_element_type=f32)
    ait1_ref[0] = _dotT(Wm1a1_ref[...], h1) + bm1c1_ref[...]


def _c_body(h1_ref, accT1_ref, cnt_ref, part_ref, pred_ref,
            Wm21_ref, bm21_ref, Wu1a1_ref, Wu1b1_ref, bu11_ref, Wu21_ref,
            bu21_ref, Wc1a_ref, Wd_ref, bd_ref, out_ref):
    f32 = jnp.float32
    h2 = _layer_update(h1_ref[0], accT1_ref[0], cnt_ref[0], Wm21_ref, bm21_ref,
                       Wu1a1_ref, Wu1b1_ref, bu11_ref, Wu21_ref, bu21_ref)
    hcat = jnp.dot(h2, Wc1a_ref[...], preferred_element_type=f32) + part_ref[0]
    out_ref[0] = (jnp.dot(jnp.maximum(hcat, 0.0), Wd_ref[...],
                          preferred_element_type=f32)
                  + bd_ref[...] + pred_ref[0])


def _bspec(shape):
    return pl.BlockSpec(shape, lambda b: (b,) + (0,) * (len(shape) - 1))


def _wspec(shape):
    return pl.BlockSpec(shape, lambda b: (0,) * len(shape))


_SMEM_BETA = pl.BlockSpec((1, 1, 1), lambda b: (b, 0, 0),
                          memory_space=pltpu.SMEM)


def _tc_call(body, in_specs, out_specs, out_shapes, args):
    return pl.pallas_call(
        body, grid=(B,), in_specs=in_specs, out_specs=out_specs,
        out_shape=out_shapes)(*args)


@jax.jit
def _run(beta2, x, ped, maskf, selff, posx, posy, idxt, maskt, hist2,
         Wef, Wet, bemb,
         Wm1a, Wm1b, bm1c, wrow, Wm2, bm2r, Wu1a, Wu1b, bu1r, Wu2, bu2r,
         Whebd, bhebd, WxT, WhT, bg, Wlo, blo, Wsp, bsp,
         Wc1a, Wc1b, Wc1c, Wc1t, bc1, Wd, bd):
    f32 = jnp.float32
    W = lambda s: _wspec(s)

    h0, ajt0, ait0, cnt = _tc_call(
        _a1_body,
        [_SMEM_BETA, _bspec((1, N, 6)), _bspec((1, N, K)),
         W((6, HID)), W((3, HID)), W((1, HID)),
         W((HID, HID)), W((HID, HID)), W((HID, 1))],
        [_bspec((1, N, HID)), _bspec((1, N, HID)), _bspec((1, HID, N)),
         _bspec((1, N, 1))],
        [jax.ShapeDtypeStruct((B, N, HID), f32),
         jax.ShapeDtypeStruct((B, N, HID), f32),
         jax.ShapeDtypeStruct((B, HID, N), f32),
         jax.ShapeDtypeStruct((B, N, 1), f32)],
        [beta2, ped, maskf, Wef, Wet, bemb, Wm1a[0], Wm1b[0], bm1c[0]])

    part, pred = _tc_call(
        _a2_body,
        [_SMEM_BETA, _bspec((1, N, 2)), _bspec((1, N, 5)),
         _bspec((1, N, T * HD)),
         W((T * HD, T * HE)), W((1, T * HE)),
         W((HE, 4 * HL)), W((HL, 4 * HL)), W((1, 4 * HL)),
         W((HL, HLO)), W((1, HLO)), W((2, SP)), W((1, SP)),
         W((HLO, SP // 2)), W((SP, SP // 2)), W((3, SP // 2)),
         W((1, SP // 2))],
        [_bspec((1, N, SP // 2)), _bspec((1, N, 2))],
        [jax.ShapeDtypeStruct((B, N, SP // 2), f32),
         jax.ShapeDtypeStruct((B, N, 2), f32)],
        [beta2, x, selff, hist2, Whebd, bhebd, WxT, WhT, bg, Wlo, blo,
         Wsp, bsp, Wc1b, Wc1c, Wc1t, bc1])

    accT0 = _sc_edges(ajt0, ait0, posx, posy, idxt, maskt, wrow[0])

    h1, ajt1, ait1 = _tc_call(
        _b_body,
        [_bspec((1, N, HID)), _bspec((1, HID, N)), _bspec((1, N, 1)),
         W((HID, HID)), W((1, HID)), W((HID, HID)), W((HID, HID)),
         W((1, HID)), W((HID, HID)), W((1, HID)),
         W((HID, HID)), W((HID, HID)), W((HID, 1))],
        [_bspec((1, N, HID)), _bspec((1, N, HID)), _bspec((1, HID, N))],
        [jax.ShapeDtypeStruct((B, N, HID), f32),
         jax.ShapeDtypeStruct((B, N, HID), f32),
         jax.ShapeDtypeStruct((B, HID, N), f32)],
        [h0, accT0, cnt, Wm2[0], bm2r[0], Wu1a[0], Wu1b[0], bu1r[0],
         Wu2[0], bu2r[0], Wm1a[1], Wm1b[1], bm1c[1]])

    accT1 = _sc_edges(ajt1, ait1, posx, posy, idxt, maskt, wrow[1])

    out = _tc_call(
        _c_body,
        [_bspec((1, N, HID)), _bspec((1, HID, N)), _bspec((1, N, 1)),
         _bspec((1, N, SP // 2)), _bspec((1, N, 2)),
         W((HID, HID)), W((1, HID)), W((HID, HID)), W((HID, HID)),
         W((1, HID)), W((HID, HID)), W((1, HID)),
         W((HID, SP // 2)), W((SP // 2, 2)), W((1, 2))],
        [_bspec((1, N, 2))],
        [jax.ShapeDtypeStruct((B, N, 2), f32)],
        [h1, accT1, cnt, part, pred, Wm2[1], bm2r[1], Wu1a[1], Wu1b[1],
         bu1r[1], Wu2[1], bu2r[1], Wc1a, Wd, bd])
    return out[0] if isinstance(out, (list, tuple)) else out


def kernel(x, beta, ped_features, neigh_ped_mask, self_features, near_ped_idx,
           hist_feature, nei_list, t, W_emb, b_emb, Wm1, bm1, Wm2, bm2,
           Wu1, bu1, Wu2, bu2, W_sp, b_sp, W_he, b_he, Wih, Whh, bih, bhh,
           W_lo, b_lo, W_c1, b_c1, W_d, b_d):
    f32 = jnp.float32
    beta2 = beta.reshape(B, 1, 1).astype(f32)
    maskf = neigh_ped_mask.astype(f32)
    idx = near_ped_idx.astype(jnp.int32)
    idxt = jnp.swapaxes(idx, 1, 2)
    maskt = jnp.swapaxes(maskf, 1, 2)
    posx = ped_features[:, :, 0].astype(f32)
    posy = ped_features[:, :, 1].astype(f32)
    hist2 = hist_feature.reshape(B, N, T * HD).astype(f32)

    Wef = W_emb[:6].astype(f32)
    Wet = W_emb[6:9].astype(f32)
    bemb = b_emb.reshape(1, HID).astype(f32)
    Wm1a = Wm1[:, :HID, :].astype(f32)
    Wm1b = Wm1[:, HID:2 * HID, :].astype(f32)
    bm1c = bm1.reshape(L, HID, 1).astype(f32)
    wrow = jnp.broadcast_to(
        Wm1[:, 2 * HID, :].astype(f32)[:, :, None], (L, HID, LN))
    Wm2c = Wm2.astype(f32)
    bm2r = bm2.reshape(L, 1, HID).astype(f32)
    Wu1a = Wu1[:, :HID, :].astype(f32)
    Wu1b = Wu1[:, HID:, :].astype(f32)
    bu1r = bu1.reshape(L, 1, HID).astype(f32)
    Wu2c = Wu2.astype(f32)
    bu2r = bu2.reshape(L, 1, HID).astype(f32)
    Whebd = jnp.zeros((T * HD, T * HE), f32)
    for tt in range(T):
        Whebd = Whebd.at[tt * HD:(tt + 1) * HD, tt * HE:(tt + 1) * HE].set(
            W_he.astype(f32))
    bhebd = jnp.tile(b_he.astype(f32), T).reshape(1, T * HE)
    WxT = Wih.T.astype(f32)
    WhT = Whh.T.astype(f32)
    bg = (bih + bhh).reshape(1, 4 * HL).astype(f32)
    Wlo = W_lo.astype(f32)
    blo = b_lo.reshape(1, HLO).astype(f32)
    Wsp = W_sp.astype(f32)
    bsp = b_sp.reshape(1, SP).astype(f32)
    Wc1a = W_c1[0:HID].astype(f32)
    Wc1b = W_c1[HID:HID + HLO].astype(f32)
    Wc1c = W_c1[HID + HLO:HID + HLO + SP].astype(f32)
    Wc1t = W_c1[HID + HLO + SP:].astype(f32)
    bc1 = b_c1.reshape(1, SP // 2).astype(f32)
    Wd = W_d.astype(f32)
    bd = b_d.reshape(1, 2).astype(f32)

    return _run(beta2, x.astype(f32), ped_features.astype(f32), maskf,
                self_features.astype(f32), posx, posy, idxt, maskt, hist2,
                Wef, Wet, bemb,
                Wm1a, Wm1b, bm1c, wrow, Wm2c, bm2r, Wu1a, Wu1b, bu1r,
                Wu2c, bu2r,
                Whebd, bhebd, WxT, WhT, bg, Wlo, blo, Wsp, bsp,
                Wc1a, Wc1b, Wc1c, Wc1t, bc1, Wd, bd)


# speedup vs baseline: 1.0809x; 1.0809x over previous
import functools
import jax
import jax.numpy as jnp
from jax import lax
from jax.experimental import pallas as pl
from jax.experimental.pallas import tpu as pltpu
from jax.experimental.pallas import tpu_sc as plsc

B, N, K, T = 64, 128, 32, 8
HID, SP, HD, HE, HL, HLO = 64, 64, 2, 32, 48, 32
L = 2
TAU = 2.0
E = N * K
NC, NS, LN = 2, 16, 16
NW = NC * NS
BPW = B // NW
NB = N // LN
DC = HID // LN


CD = 8
NCH = HID // CD


def _sc_edges_body(ajt, ait, posx, posy, idxt, maskt, wrow, accT,
                   tab_v, ai_v, px_v, py_v, idx_v, msk_v, wr_v, acc_v, d2_v):
    wid = lax.axis_index("s") * NC + lax.axis_index("c")
    pltpu.sync_copy(wrow, wr_v)
    for bb in range(BPW):
        b = wid * BPW + bb
        pltpu.sync_copy(ajt.at[b], tab_v)
        pltpu.sync_copy(ait.at[b], ai_v)
        pltpu.sync_copy(posx.at[b], px_v)
        pltpu.sync_copy(posy.at[b], py_v)
        pltpu.sync_copy(idxt.at[b], idx_v)
        pltpu.sync_copy(maskt.at[b], msk_v)

        def blk_body(blk, carry_blk):
            off = blk * LN
            pxi = px_v[pl.ds(off, LN)]
            pyi = py_v[pl.ds(off, LN)]

            def pre_body(k, _):
                jv = idx_v[k, pl.ds(off, LN)]
                gx = plsc.load_gather(px_v, [jv])
                gy = plsc.load_gather(py_v, [jv])
                rx = pxi - gx
                ry = pyi - gy
                d2_v[k, :] = rx * rx + ry * ry
                return _

            plsc.parallel_loop(0, K, 1, unroll=4, carry=jnp.int32(0))(pre_body)

            for c in range(NCH):
                ai_c = [ai_v[c * CD + d, pl.ds(off, LN)] for d in range(CD)]
                wr_c = [wr_v[c * CD + d, :] for d in range(CD)]

                def k_body(k, carry):
                    jv = idx_v[k, pl.ds(off, LN)]
                    mv = msk_v[k, pl.ds(off, LN)]
                    d2 = d2_v[k, :]
                    out = []
                    for d in range(CD):
                        dd = c * CD + d
                        g = plsc.load_gather(
                            tab_v, [jv, jnp.full((LN,), dd, jnp.int32)])
                        v = jnp.maximum(ai_c[d] + g + d2 * wr_c[d], 0.0)
                        out.append(carry[d] + mv * v)
                    return tuple(out)

                acc0 = tuple(jnp.zeros((LN,), jnp.float32) for _ in range(CD))
                acc = plsc.parallel_loop(0, K, 1, unroll=2, carry=acc0)(k_body)
                for d in range(CD):
                    acc_v[c * CD + d, pl.ds(off, LN)] = acc[d]
            return carry_blk

        lax.fori_loop(0, NB, blk_body, 0)
        pltpu.sync_copy(acc_v, accT.at[b])


def _sc_edges(ajt, ait, posx, posy, idxt, maskt, wrow):
    mesh = plsc.VectorSubcoreMesh(core_axis_name="c", subcore_axis_name="s")
    return pl.kernel(
        _sc_edges_body,
        mesh=mesh,
        compiler_params=pltpu.CompilerParams(needs_layout_passes=False),
        out_type=jax.ShapeDtypeStruct((B, HID, N), jnp.float32),
        scratch_types=[
            pltpu.VMEM((N, HID), jnp.float32),
            pltpu.VMEM((HID, N), jnp.float32),
            pltpu.VMEM((N,), jnp.float32),
            pltpu.VMEM((N,), jnp.float32),
            pltpu.VMEM((K, N), jnp.int32),
            pltpu.VMEM((K, N), jnp.float32),
            pltpu.VMEM((HID, LN), jnp.float32),
            pltpu.VMEM((HID, N), jnp.float32),
            pltpu.VMEM((K, LN), jnp.float32),
        ],
    )(ajt, ait, posx, posy, idxt, maskt, wrow)


def _dotT(w, h):
    return lax.dot_general(w, h, (((0,), (1,)), ((), ())),
                           preferred_element_type=jnp.float32)


def _a1_body(beta_ref, ped_ref, maskf_ref,
             Wef_ref, Wet_ref, bemb_ref, Wm1a0_ref, Wm1b0_ref, bm1c0_ref,
             h0_ref, ajt0_ref, ait0_ref, cnt_ref):
    f32 = jnp.float32
    beta = beta_ref[0, 0, 0]
    bvec = jnp.full((1, 1), beta, f32)
    sb = jnp.sin(bvec)
    cb = jnp.cos(bvec)
    trow = (beta * Wet_ref[0:1, :] + sb * Wet_ref[1:2, :]
            + cb * Wet_ref[2:3, :] + bemb_ref[...])
    ped = ped_ref[0]
    h = jnp.dot(ped, Wef_ref[...], preferred_element_type=f32) + trow
    h0_ref[0] = h
    ajt0_ref[0] = jnp.dot(h, Wm1b0_ref[...], preferred_element_type=f32)
    ait0_ref[0] = _dotT(Wm1a0_ref[...], h) + bm1c0_ref[...]
    cnt_ref[0] = jnp.sum(maskf_ref[0], axis=1, keepdims=True)


def _a2_body(beta_ref, x_ref, self_ref, hist_ref,
             Whebd_ref, bhebd_ref, WxT_ref, WhT_ref, bg_ref, Wlo_ref, blo_ref,
             Wsp_ref, bsp_ref, Wc1b_ref, Wc1c_ref, Wc1t_ref, bc1_ref,
             part_ref, pred_ref):
    f32 = jnp.float32
    beta = beta_ref[0, 0, 0]
    bvec = jnp.full((1, 1), beta, f32)
    sb = jnp.sin(bvec)
    cb = jnp.cos(bvec)

    he = jnp.maximum(
        jnp.dot(hist_ref[0], Whebd_ref[...], preferred_element_type=f32)
        + bhebd_ref[...], 0.0)
    hs = jnp.zeros((N, HL), f32)
    cs = jnp.zeros((N, HL), f32)
    for tt in range(T):
        xt = he[:, HE * tt:HE * (tt + 1)]
        gates = (jnp.dot(xt, WxT_ref[...], preferred_element_type=f32)
                 + jnp.dot(hs, WhT_ref[...], preferred_element_type=f32)
                 + bg_ref[...])
        ig = jax.nn.sigmoid(gates[:, 0:HL])
        fg = jax.nn.sigmoid(gates[:, HL:2 * HL])
        gg = jnp.tanh(gates[:, 2 * HL:3 * HL])
        og = jax.nn.sigmoid(gates[:, 3 * HL:4 * HL])
        cs = fg * cs + ig * gg
        hs = og * jnp.tanh(cs)
    hist_out = jnp.dot(hs, Wlo_ref[...], preferred_element_type=f32) + blo_ref[...]

    spatial = jnp.maximum(
        jnp.dot(x_ref[0], Wsp_ref[...], preferred_element_type=f32)
        + bsp_ref[...], 0.0)
    sf = self_ref[0]
    ds = sf[:, 4:5]
    sx = sf[:, 0:1]
    sy = sf[:, 1:2]
    temp = jnp.sqrt(sx * sx + sy * sy)
    temp_ = jnp.where(temp == 0.0, temp + 0.1, temp)
    predx = (ds * sx / temp_ - sf[:, 2:3]) / TAU
    predy = (ds * sy / temp_ - sf[:, 3:4]) / TAU
    pred_ref[0] = jnp.concatenate([predx, predy], axis=1)

    trow2 = (beta * Wc1t_ref[0:1, :] + sb * Wc1t_ref[1:2, :]
             + cb * Wc1t_ref[2:3, :] + bc1_ref[...])
    part_ref[0] = (jnp.dot(hist_out, Wc1b_ref[...], preferred_element_type=f32)
                   + jnp.dot(spatial, Wc1c_ref[...], preferred_element_type=f32)
                   + trow2)


def _layer_update(h, accT, cnt, Wm2_ref, bm2_ref, Wu1a_ref, Wu1b_ref,
                  bu1_ref, Wu2_ref, bu2_ref):
    f32 = jnp.float32
    agg = (lax.dot_general(accT, Wm2_ref[...], (((0,), (0,)), ((), ())),
                           preferred_element_type=f32)
           + cnt * bm2_ref[...])
    upd = jnp.maximum(
        jnp.dot(h, Wu1a_ref[...], preferred_element_type=f32)
        + jnp.dot(agg, Wu1b_ref[...], preferred_element_type=f32)
        + bu1_ref[...], 0.0)
    return h + jnp.dot(upd, Wu2_ref[...], preferred_element_type=f32) + bu2_ref[...]


def _b_body(h0_ref, accT0_ref, cnt_ref,
            Wm20_ref, bm20_ref, Wu1a0_ref, Wu1b0_ref, bu10_ref, Wu20_ref,
            bu20_ref, Wm1a1_ref, Wm1b1_ref, bm1c1_ref,
            h1_ref, ajt1_ref, ait1_ref):
    f32 = jnp.float32
    h1 = _layer_update(h0_ref[0], accT0_ref[0], cnt_ref[0], Wm20_ref, bm20_ref,
                       Wu1a0_ref, Wu1b0_ref, bu10_ref, Wu20_ref, bu20_ref)
    h1_ref[0] = h1
    ajt1_ref[0] = jnp.dot(h1, Wm1b1_ref[...], preferred_element_type=f32)
    ait1_ref[0] = _dotT(Wm1a1_ref[...], h1) + bm1c1_ref[...]


def _c_body(h1_ref, accT1_ref, cnt_ref, part_ref, pred_ref,
            Wm21_ref, bm21_ref, Wu1a1_ref, Wu1b1_ref, bu11_ref, Wu21_ref,
            bu21_ref, Wc1a_ref, Wd_ref, bd_ref, out_ref):
    f32 = jnp.float32
    h2 = _layer_update(h1_ref[0], accT1_ref[0], cnt_ref[0], Wm21_ref, bm21_ref,
                       Wu1a1_ref, Wu1b1_ref, bu11_ref, Wu21_ref, bu21_ref)
    hcat = jnp.dot(h2, Wc1a_ref[...], preferred_element_type=f32) + part_ref[0]
    out_ref[0] = (jnp.dot(jnp.maximum(hcat, 0.0), Wd_ref[...],
                          preferred_element_type=f32)
                  + bd_ref[...] + pred_ref[0])


def _bspec(shape):
    return pl.BlockSpec(shape, lambda b: (b,) + (0,) * (len(shape) - 1))


def _wspec(shape):
    return pl.BlockSpec(shape, lambda b: (0,) * len(shape))


_SMEM_BETA = pl.BlockSpec((1, 1, 1), lambda b: (b, 0, 0),
                          memory_space=pltpu.SMEM)


def _tc_call(body, in_specs, out_specs, out_shapes, args):
    return pl.pallas_call(
        body, grid=(B,), in_specs=in_specs, out_specs=out_specs,
        out_shape=out_shapes)(*args)


@jax.jit
def _run(beta2, x, ped, maskf, selff, posx, posy, idxt, maskt, hist2,
         Wef, Wet, bemb,
         Wm1a, Wm1b, bm1c, wrow, Wm2, bm2r, Wu1a, Wu1b, bu1r, Wu2, bu2r,
         Whebd, bhebd, WxT, WhT, bg, Wlo, blo, Wsp, bsp,
         Wc1a, Wc1b, Wc1c, Wc1t, bc1, Wd, bd):
    f32 = jnp.float32
    W = lambda s: _wspec(s)

    h0, ajt0, ait0, cnt = _tc_call(
        _a1_body,
        [_SMEM_BETA, _bspec((1, N, 6)), _bspec((1, N, K)),
         W((6, HID)), W((3, HID)), W((1, HID)),
         W((HID, HID)), W((HID, HID)), W((HID, 1))],
        [_bspec((1, N, HID)), _bspec((1, N, HID)), _bspec((1, HID, N)),
         _bspec((1, N, 1))],
        [jax.ShapeDtypeStruct((B, N, HID), f32),
         jax.ShapeDtypeStruct((B, N, HID), f32),
         jax.ShapeDtypeStruct((B, HID, N), f32),
         jax.ShapeDtypeStruct((B, N, 1), f32)],
        [beta2, ped, maskf, Wef, Wet, bemb, Wm1a[0], Wm1b[0], bm1c[0]])

    part, pred = _tc_call(
        _a2_body,
        [_SMEM_BETA, _bspec((1, N, 2)), _bspec((1, N, 5)),
         _bspec((1, N, T * HD)),
         W((T * HD, T * HE)), W((1, T * HE)),
         W((HE, 4 * HL)), W((HL, 4 * HL)), W((1, 4 * HL)),
         W((HL, HLO)), W((1, HLO)), W((2, SP)), W((1, SP)),
         W((HLO, SP // 2)), W((SP, SP // 2)), W((3, SP // 2)),
         W((1, SP // 2))],
        [_bspec((1, N, SP // 2)), _bspec((1, N, 2))],
        [jax.ShapeDtypeStruct((B, N, SP // 2), f32),
         jax.ShapeDtypeStruct((B, N, 2), f32)],
        [beta2, x, selff, hist2, Whebd, bhebd, WxT, WhT, bg, Wlo, blo,
         Wsp, bsp, Wc1b, Wc1c, Wc1t, bc1])

    accT0 = _sc_edges(ajt0, ait0, posx, posy, idxt, maskt, wrow[0])

    h1, ajt1, ait1 = _tc_call(
        _b_body,
        [_bspec((1, N, HID)), _bspec((1, HID, N)), _bspec((1, N, 1)),
         W((HID, HID)), W((1, HID)), W((HID, HID)), W((HID, HID)),
         W((1, HID)), W((HID, HID)), W((1, HID)),
         W((HID, HID)), W((HID, HID)), W((HID, 1))],
        [_bspec((1, N, HID)), _bspec((1, N, HID)), _bspec((1, HID, N))],
        [jax.ShapeDtypeStruct((B, N, HID), f32),
         jax.ShapeDtypeStruct((B, N, HID), f32),
         jax.ShapeDtypeStruct((B, HID, N), f32)],
        [h0, accT0, cnt, Wm2[0], bm2r[0], Wu1a[0], Wu1b[0], bu1r[0],
         Wu2[0], bu2r[0], Wm1a[1], Wm1b[1], bm1c[1]])

    accT1 = _sc_edges(ajt1, ait1, posx, posy, idxt, maskt, wrow[1])

    out = _tc_call(
        _c_body,
        [_bspec((1, N, HID)), _bspec((1, HID, N)), _bspec((1, N, 1)),
         _bspec((1, N, SP // 2)), _bspec((1, N, 2)),
         W((HID, HID)), W((1, HID)), W((HID, HID)), W((HID, HID)),
         W((1, HID)), W((HID, HID)), W((1, HID)),
         W((HID, SP // 2)), W((SP // 2, 2)), W((1, 2))],
        [_bspec((1, N, 2))],
        [jax.ShapeDtypeStruct((B, N, 2), f32)],
        [h1, accT1, cnt, part, pred, Wm2[1], bm2r[1], Wu1a[1], Wu1b[1],
         bu1r[1], Wu2[1], bu2r[1], Wc1a, Wd, bd])
    return out[0] if isinstance(out, (list, tuple)) else out


def kernel(x, beta, ped_features, neigh_ped_mask, self_features, near_ped_idx,
           hist_feature, nei_list, t, W_emb, b_emb, Wm1, bm1, Wm2, bm2,
           Wu1, bu1, Wu2, bu2, W_sp, b_sp, W_he, b_he, Wih, Whh, bih, bhh,
           W_lo, b_lo, W_c1, b_c1, W_d, b_d):
    f32 = jnp.float32
    beta2 = beta.reshape(B, 1, 1).astype(f32)
    maskf = neigh_ped_mask.astype(f32)
    idx = near_ped_idx.astype(jnp.int32)
    idxt = jnp.swapaxes(idx, 1, 2)
    maskt = jnp.swapaxes(maskf, 1, 2)
    posx = ped_features[:, :, 0].astype(f32)
    posy = ped_features[:, :, 1].astype(f32)
    hist2 = hist_feature.reshape(B, N, T * HD).astype(f32)

    Wef = W_emb[:6].astype(f32)
    Wet = W_emb[6:9].astype(f32)
    bemb = b_emb.reshape(1, HID).astype(f32)
    Wm1a = Wm1[:, :HID, :].astype(f32)
    Wm1b = Wm1[:, HID:2 * HID, :].astype(f32)
    bm1c = bm1.reshape(L, HID, 1).astype(f32)
    wrow = jnp.broadcast_to(
        Wm1[:, 2 * HID, :].astype(f32)[:, :, None], (L, HID, LN))
    Wm2c = Wm2.astype(f32)
    bm2r = bm2.reshape(L, 1, HID).astype(f32)
    Wu1a = Wu1[:, :HID, :].astype(f32)
    Wu1b = Wu1[:, HID:, :].astype(f32)
    bu1r = bu1.reshape(L, 1, HID).astype(f32)
    Wu2c = Wu2.astype(f32)
    bu2r = bu2.reshape(L, 1, HID).astype(f32)
    Whebd = jnp.zeros((T * HD, T * HE), f32)
    for tt in range(T):
        Whebd = Whebd.at[tt * HD:(tt + 1) * HD, tt * HE:(tt + 1) * HE].set(
            W_he.astype(f32))
    bhebd = jnp.tile(b_he.astype(f32), T).reshape(1, T * HE)
    WxT = Wih.T.astype(f32)
    WhT = Whh.T.astype(f32)
    bg = (bih + bhh).reshape(1, 4 * HL).astype(f32)
    Wlo = W_lo.astype(f32)
    blo = b_lo.reshape(1, HLO).astype(f32)
    Wsp = W_sp.astype(f32)
    bsp = b_sp.reshape(1, SP).astype(f32)
    Wc1a = W_c1[0:HID].astype(f32)
    Wc1b = W_c1[HID:HID + HLO].astype(f32)
    Wc1c = W_c1[HID + HLO:HID + HLO + SP].astype(f32)
    Wc1t = W_c1[HID + HLO + SP:].astype(f32)
    bc1 = b_c1.reshape(1, SP // 2).astype(f32)
    Wd = W_d.astype(f32)
    bd = b_d.reshape(1, 2).astype(f32)

    return _run(beta2, x.astype(f32), ped_features.astype(f32), maskf,
                self_features.astype(f32), posx, posy, idxt, maskt, hist2,
                Wef, Wet, bemb,
                Wm1a, Wm1b, bm1c, wrow, Wm2c, bm2r, Wu1a, Wu1b, bu1r,
                Wu2c, bu2r,
                Whebd, bhebd, WxT, WhT, bg, Wlo, blo, Wsp, bsp,
                Wc1a, Wc1b, Wc1c, Wc1t, bc1, Wd, bd)


# speedup vs baseline: 1.2976x; 1.2005x over previous
import functools
import jax
import jax.numpy as jnp
from jax import lax
from jax.experimental import pallas as pl
from jax.experimental.pallas import tpu as pltpu
from jax.experimental.pallas import tpu_sc as plsc

B, N, K, T = 64, 128, 32, 8
HID, SP, HD, HE, HL, HLO = 64, 64, 2, 32, 48, 32
L = 2
TAU = 2.0
E = N * K
NC, NS, LN = 2, 16, 16
NW = NC * NS
BPW = B // NW
NB = N // LN
DC = HID // LN


EB = LN * K
NCH = HID // LN


def _sc_edges_body(tabflat, ai, posx, posy, gidx, idxl, maskl, wrow, acc,
                   rows_v, ai_v, px_v, py_v, gi_v, jl_v, mk_v, wr_v, acc_v,
                   d2_v, sem):
    i32 = jnp.int32
    wid = lax.axis_index("s") * NC + lax.axis_index("c")
    pltpu.sync_copy(wrow, wr_v)
    wr_c = [wr_v[pl.ds(c * LN, LN)] for c in range(NCH)]
    for bb in range(BPW):
        b = wid * BPW + bb
        pltpu.sync_copy(ai.at[b], ai_v)
        pltpu.sync_copy(posx.at[b], px_v)
        pltpu.sync_copy(posy.at[b], py_v)

        def blk_body(blk, carry_blk):
            pltpu.sync_copy(gidx.at[b, pl.ds(blk * 4, 4)], gi_v)
            pltpu.sync_copy(idxl.at[b, pl.ds(blk * EB, EB)], jl_v)
            pltpu.sync_copy(maskl.at[b, pl.ds(blk * EB, EB)], mk_v)
            cps = [pltpu.async_copy(tabflat.at[gi_v.at[j]],
                                    rows_v.at[pl.ds(j * 128, 128)], sem)
                   for j in range(4)]

            def pre_body(g, _):
                node = blk * LN + g // 2
                nsplat = jnp.full((LN,), node, i32)
                jv = jl_v[pl.ds(g * LN, LN)]
                gx = plsc.load_gather(px_v, [jv])
                gy = plsc.load_gather(py_v, [jv])
                pxi = plsc.load_gather(px_v, [nsplat])
                pyi = plsc.load_gather(py_v, [nsplat])
                rx = pxi - gx
                ry = pyi - gy
                d2_v[pl.ds(g * LN, LN)] = rx * rx + ry * ry
                return _

            plsc.parallel_loop(0, EB // LN, 1, unroll=4,
                               carry=jnp.int32(0))(pre_body)
            for cp in cps:
                cp.wait()

            for il in range(LN):
                node = blk * LN + il
                ai_c = [ai_v[node, pl.ds(c * LN, LN)] for c in range(NCH)]

                def k_body(k, carry):
                    e = il * K + k
                    esplat = jnp.full((LN,), e, i32)
                    sd2 = plsc.load_gather(d2_v, [esplat])
                    smv = plsc.load_gather(mk_v, [esplat])
                    out = []
                    for c in range(NCH):
                        row = rows_v[e, pl.ds(c * LN, LN)]
                        v = jnp.maximum(ai_c[c] + row + sd2 * wr_c[c], 0.0)
                        out.append(carry[c] + smv * v)
                    return tuple(out)

                acc0 = tuple(jnp.zeros((LN,), jnp.float32) for _ in range(NCH))
                accs = plsc.parallel_loop(0, K, 1, unroll=2, carry=acc0)(k_body)
                for c in range(NCH):
                    acc_v[node, pl.ds(c * LN, LN)] = accs[c]
            return carry_blk

        lax.fori_loop(0, NB, blk_body, 0)
        pltpu.sync_copy(acc_v, acc.at[b])


def _sc_edges(tabflat, ai, posx, posy, gidx, idxl, maskl, wrow):
    mesh = plsc.VectorSubcoreMesh(core_axis_name="c", subcore_axis_name="s")
    return pl.kernel(
        _sc_edges_body,
        mesh=mesh,
        compiler_params=pltpu.CompilerParams(needs_layout_passes=False,
                                             use_tc_tiling_on_sc=False),
        out_type=jax.ShapeDtypeStruct((B, N, HID), jnp.float32),
        scratch_types=[
            pltpu.VMEM((EB, HID), jnp.float32),
            pltpu.VMEM((N, HID), jnp.float32),
            pltpu.VMEM((N,), jnp.float32),
            pltpu.VMEM((N,), jnp.float32),
            pltpu.VMEM((4, 128), jnp.int32),
            pltpu.VMEM((EB,), jnp.int32),
            pltpu.VMEM((EB,), jnp.float32),
            pltpu.VMEM((HID,), jnp.float32),
            pltpu.VMEM((N, HID), jnp.float32),
            pltpu.VMEM((EB,), jnp.float32),
            pltpu.SemaphoreType.DMA,
        ],
    )(tabflat, ai, posx, posy, gidx, idxl, maskl, wrow)


def _a1_body(beta_ref, ped_ref, maskf_ref,
             Wef_ref, Wet_ref, bemb_ref, Wm1a0_ref, Wm1b0_ref, bm1r0_ref,
             h0_ref, ajt0_ref, ai0_ref, cnt_ref):
    f32 = jnp.float32
    beta = beta_ref[0, 0, 0]
    bvec = jnp.full((1, 1), beta, f32)
    sb = jnp.sin(bvec)
    cb = jnp.cos(bvec)
    trow = (beta * Wet_ref[0:1, :] + sb * Wet_ref[1:2, :]
            + cb * Wet_ref[2:3, :] + bemb_ref[...])
    ped = ped_ref[0]
    h = jnp.dot(ped, Wef_ref[...], preferred_element_type=f32) + trow
    h0_ref[0] = h
    ajt0_ref[0] = jnp.dot(h, Wm1b0_ref[...], preferred_element_type=f32)
    ai0_ref[0] = jnp.dot(h, Wm1a0_ref[...], preferred_element_type=f32) + bm1r0_ref[...]
    cnt_ref[0] = jnp.sum(maskf_ref[0], axis=1, keepdims=True)


def _a2_body(beta_ref, x_ref, self_ref, hist_ref,
             Whebd_ref, bhebd_ref, WxT_ref, WhT_ref, bg_ref, Wlo_ref, blo_ref,
             Wsp_ref, bsp_ref, Wc1b_ref, Wc1c_ref, Wc1t_ref, bc1_ref,
             part_ref, pred_ref):
    f32 = jnp.float32
    beta = beta_ref[0, 0, 0]
    bvec = jnp.full((1, 1), beta, f32)
    sb = jnp.sin(bvec)
    cb = jnp.cos(bvec)

    he = jnp.maximum(
        jnp.dot(hist_ref[0], Whebd_ref[...], preferred_element_type=f32)
        + bhebd_ref[...], 0.0)
    hs = jnp.zeros((N, HL), f32)
    cs = jnp.zeros((N, HL), f32)
    for tt in range(T):
        xt = he[:, HE * tt:HE * (tt + 1)]
        gates = (jnp.dot(xt, WxT_ref[...], preferred_element_type=f32)
                 + jnp.dot(hs, WhT_ref[...], preferred_element_type=f32)
                 + bg_ref[...])
        ig = jax.nn.sigmoid(gates[:, 0:HL])
        fg = jax.nn.sigmoid(gates[:, HL:2 * HL])
        gg = jnp.tanh(gates[:, 2 * HL:3 * HL])
        og = jax.nn.sigmoid(gates[:, 3 * HL:4 * HL])
        cs = fg * cs + ig * gg
        hs = og * jnp.tanh(cs)
    hist_out = jnp.dot(hs, Wlo_ref[...], preferred_element_type=f32) + blo_ref[...]

    spatial = jnp.maximum(
        jnp.dot(x_ref[0], Wsp_ref[...], preferred_element_type=f32)
        + bsp_ref[...], 0.0)
    sf = self_ref[0]
    ds = sf[:, 4:5]
    sx = sf[:, 0:1]
    sy = sf[:, 1:2]
    temp = jnp.sqrt(sx * sx + sy * sy)
    temp_ = jnp.where(temp == 0.0, temp + 0.1, temp)
    predx = (ds * sx / temp_ - sf[:, 2:3]) / TAU
    predy = (ds * sy / temp_ - sf[:, 3:4]) / TAU
    pred_ref[0] = jnp.concatenate([predx, predy], axis=1)

    trow2 = (beta * Wc1t_ref[0:1, :] + sb * Wc1t_ref[1:2, :]
             + cb * Wc1t_ref[2:3, :] + bc1_ref[...])
    part_ref[0] = (jnp.dot(hist_out, Wc1b_ref[...], preferred_element_type=f32)
                   + jnp.dot(spatial, Wc1c_ref[...], preferred_element_type=f32)
                   + trow2)


def _layer_update(h, acc, cnt, Wm2_ref, bm2_ref, Wu1a_ref, Wu1b_ref,
                  bu1_ref, Wu2_ref, bu2_ref):
    f32 = jnp.float32
    agg = (jnp.dot(acc, Wm2_ref[...], preferred_element_type=f32)
           + cnt * bm2_ref[...])
    upd = jnp.maximum(
        jnp.dot(h, Wu1a_ref[...], preferred_element_type=f32)
        + jnp.dot(agg, Wu1b_ref[...], preferred_element_type=f32)
        + bu1_ref[...], 0.0)
    return h + jnp.dot(upd, Wu2_ref[...], preferred_element_type=f32) + bu2_ref[...]


def _b_body(h0_ref, acc0_ref, cnt_ref,
            Wm20_ref, bm20_ref, Wu1a0_ref, Wu1b0_ref, bu10_ref, Wu20_ref,
            bu20_ref, Wm1a1_ref, Wm1b1_ref, bm1r1_ref,
            h1_ref, ajt1_ref, ai1_ref):
    f32 = jnp.float32
    h1 = _layer_update(h0_ref[0], acc0_ref[0], cnt_ref[0], Wm20_ref, bm20_ref,
                       Wu1a0_ref, Wu1b0_ref, bu10_ref, Wu20_ref, bu20_ref)
    h1_ref[0] = h1
    ajt1_ref[0] = jnp.dot(h1, Wm1b1_ref[...], preferred_element_type=f32)
    ai1_ref[0] = jnp.dot(h1, Wm1a1_ref[...], preferred_element_type=f32) + bm1r1_ref[...]


def _c_body(h1_ref, acc1_ref, cnt_ref, part_ref, pred_ref,
            Wm21_ref, bm21_ref, Wu1a1_ref, Wu1b1_ref, bu11_ref, Wu21_ref,
            bu21_ref, Wc1a_ref, Wd_ref, bd_ref, out_ref):
    f32 = jnp.float32
    h2 = _layer_update(h1_ref[0], acc1_ref[0], cnt_ref[0], Wm21_ref, bm21_ref,
                       Wu1a1_ref, Wu1b1_ref, bu11_ref, Wu21_ref, bu21_ref)
    hcat = jnp.dot(h2, Wc1a_ref[...], preferred_element_type=f32) + part_ref[0]
    out_ref[0] = (jnp.dot(jnp.maximum(hcat, 0.0), Wd_ref[...],
                          preferred_element_type=f32)
                  + bd_ref[...] + pred_ref[0])


def _bspec(shape):
    return pl.BlockSpec(shape, lambda b: (b,) + (0,) * (len(shape) - 1))


def _wspec(shape):
    return pl.BlockSpec(shape, lambda b: (0,) * len(shape))


_SMEM_BETA = pl.BlockSpec((1, 1, 1), lambda b: (b, 0, 0),
                          memory_space=pltpu.SMEM)


def _tc_call(body, in_specs, out_specs, out_shapes, args):
    return pl.pallas_call(
        body, grid=(B,), in_specs=in_specs, out_specs=out_specs,
        out_shape=out_shapes)(*args)


@jax.jit
def _run(beta2, x, ped, maskf, selff, posx, posy, gidx, idxl, maskl, hist2,
         Wef, Wet, bemb,
         Wm1a, Wm1b, bm1r, wrow, Wm2, bm2r, Wu1a, Wu1b, bu1r, Wu2, bu2r,
         Whebd, bhebd, WxT, WhT, bg, Wlo, blo, Wsp, bsp,
         Wc1a, Wc1b, Wc1c, Wc1t, bc1, Wd, bd):
    f32 = jnp.float32
    W = lambda s: _wspec(s)

    h0, ajt0, ai0, cnt = _tc_call(
        _a1_body,
        [_SMEM_BETA, _bspec((1, N, 6)), _bspec((1, N, K)),
         W((6, HID)), W((3, HID)), W((1, HID)),
         W((HID, HID)), W((HID, HID)), W((1, HID))],
        [_bspec((1, N, HID)), _bspec((1, N, HID)), _bspec((1, N, HID)),
         _bspec((1, N, 1))],
        [jax.ShapeDtypeStruct((B, N, HID), f32),
         jax.ShapeDtypeStruct((B, N, HID), f32),
         jax.ShapeDtypeStruct((B, N, HID), f32),
         jax.ShapeDtypeStruct((B, N, 1), f32)],
        [beta2, ped, maskf, Wef, Wet, bemb, Wm1a[0], Wm1b[0], bm1r[0]])

    part, pred = _tc_call(
        _a2_body,
        [_SMEM_BETA, _bspec((1, N, 2)), _bspec((1, N, 5)),
         _bspec((1, N, T * HD)),
         W((T * HD, T * HE)), W((1, T * HE)),
         W((HE, 4 * HL)), W((HL, 4 * HL)), W((1, 4 * HL)),
         W((HL, HLO)), W((1, HLO)), W((2, SP)), W((1, SP)),
         W((HLO, SP // 2)), W((SP, SP // 2)), W((3, SP // 2)),
         W((1, SP // 2))],
        [_bspec((1, N, SP // 2)), _bspec((1, N, 2))],
        [jax.ShapeDtypeStruct((B, N, SP // 2), f32),
         jax.ShapeDtypeStruct((B, N, 2), f32)],
        [beta2, x, selff, hist2, Whebd, bhebd, WxT, WhT, bg, Wlo, blo,
         Wsp, bsp, Wc1b, Wc1c, Wc1t, bc1])

    acc0 = _sc_edges(ajt0.reshape(B * N, HID), ai0, posx, posy,
                     gidx, idxl, maskl, wrow[0])

    h1, ajt1, ai1 = _tc_call(
        _b_body,
        [_bspec((1, N, HID)), _bspec((1, N, HID)), _bspec((1, N, 1)),
         W((HID, HID)), W((1, HID)), W((HID, HID)), W((HID, HID)),
         W((1, HID)), W((HID, HID)), W((1, HID)),
         W((HID, HID)), W((HID, HID)), W((1, HID))],
        [_bspec((1, N, HID)), _bspec((1, N, HID)), _bspec((1, N, HID))],
        [jax.ShapeDtypeStruct((B, N, HID), f32),
         jax.ShapeDtypeStruct((B, N, HID), f32),
         jax.ShapeDtypeStruct((B, N, HID), f32)],
        [h0, acc0, cnt, Wm2[0], bm2r[0], Wu1a[0], Wu1b[0], bu1r[0],
         Wu2[0], bu2r[0], Wm1a[1], Wm1b[1], bm1r[1]])

    acc1 = _sc_edges(ajt1.reshape(B * N, HID), ai1, posx, posy,
                     gidx, idxl, maskl, wrow[1])

    out = _tc_call(
        _c_body,
        [_bspec((1, N, HID)), _bspec((1, N, HID)), _bspec((1, N, 1)),
         _bspec((1, N, SP // 2)), _bspec((1, N, 2)),
         W((HID, HID)), W((1, HID)), W((HID, HID)), W((HID, HID)),
         W((1, HID)), W((HID, HID)), W((1, HID)),
         W((HID, SP // 2)), W((SP // 2, 2)), W((1, 2))],
        [_bspec((1, N, 2))],
        [jax.ShapeDtypeStruct((B, N, 2), f32)],
        [h1, acc1, cnt, part, pred, Wm2[1], bm2r[1], Wu1a[1], Wu1b[1],
         bu1r[1], Wu2[1], bu2r[1], Wc1a, Wd, bd])
    return out[0] if isinstance(out, (list, tuple)) else out


def kernel(x, beta, ped_features, neigh_ped_mask, self_features, near_ped_idx,
           hist_feature, nei_list, t, W_emb, b_emb, Wm1, bm1, Wm2, bm2,
           Wu1, bu1, Wu2, bu2, W_sp, b_sp, W_he, b_he, Wih, Whh, bih, bhh,
           W_lo, b_lo, W_c1, b_c1, W_d, b_d):
    f32 = jnp.float32
    beta2 = beta.reshape(B, 1, 1).astype(f32)
    maskf = neigh_ped_mask.astype(f32)
    idx = near_ped_idx.astype(jnp.int32)
    idxl = idx.reshape(B, E)
    gidx = (idx + (jnp.arange(B, dtype=jnp.int32) * N)[:, None, None]
            ).reshape(B, E // 128, 128)
    maskl = maskf.reshape(B, E)
    posx = ped_features[:, :, 0].astype(f32)
    posy = ped_features[:, :, 1].astype(f32)
    hist2 = hist_feature.reshape(B, N, T * HD).astype(f32)

    Wef = W_emb[:6].astype(f32)
    Wet = W_emb[6:9].astype(f32)
    bemb = b_emb.reshape(1, HID).astype(f32)
    Wm1a = Wm1[:, :HID, :].astype(f32)
    Wm1b = Wm1[:, HID:2 * HID, :].astype(f32)
    bm1r = bm1.reshape(L, 1, HID).astype(f32)
    wrow = Wm1[:, 2 * HID, :].astype(f32)
    Wm2c = Wm2.astype(f32)
    bm2r = bm2.reshape(L, 1, HID).astype(f32)
    Wu1a = Wu1[:, :HID, :].astype(f32)
    Wu1b = Wu1[:, HID:, :].astype(f32)
    bu1r = bu1.reshape(L, 1, HID).astype(f32)
    Wu2c = Wu2.astype(f32)
    bu2r = bu2.reshape(L, 1, HID).astype(f32)
    Whebd = jnp.zeros((T * HD, T * HE), f32)
    for tt in range(T):
        Whebd = Whebd.at[tt * HD:(tt + 1) * HD, tt * HE:(tt + 1) * HE].set(
            W_he.astype(f32))
    bhebd = jnp.tile(b_he.astype(f32), T).reshape(1, T * HE)
    WxT = Wih.T.astype(f32)
    WhT = Whh.T.astype(f32)
    bg = (bih + bhh).reshape(1, 4 * HL).astype(f32)
    Wlo = W_lo.astype(f32)
    blo = b_lo.reshape(1, HLO).astype(f32)
    Wsp = W_sp.astype(f32)
    bsp = b_sp.reshape(1, SP).astype(f32)
    Wc1a = W_c1[0:HID].astype(f32)
    Wc1b = W_c1[HID:HID + HLO].astype(f32)
    Wc1c = W_c1[HID + HLO:HID + HLO + SP].astype(f32)
    Wc1t = W_c1[HID + HLO + SP:].astype(f32)
    bc1 = b_c1.reshape(1, SP // 2).astype(f32)
    Wd = W_d.astype(f32)
    bd = b_d.reshape(1, 2).astype(f32)

    return _run(beta2, x.astype(f32), ped_features.astype(f32), maskf,
                self_features.astype(f32), posx, posy, gidx, idxl, maskl,
                hist2, Wef, Wet, bemb,
                Wm1a, Wm1b, bm1r, wrow, Wm2c, bm2r, Wu1a, Wu1b, bu1r,
                Wu2c, bu2r,
                Whebd, bhebd, WxT, WhT, bg, Wlo, blo, Wsp, bsp,
                Wc1a, Wc1b, Wc1c, Wc1t, bc1, Wd, bd)


# speedup vs baseline: 1.3002x; 1.0020x over previous
import functools
import jax
import jax.numpy as jnp
from jax import lax
from jax.experimental import pallas as pl
from jax.experimental.pallas import tpu as pltpu
from jax.experimental.pallas import tpu_sc as plsc

B, N, K, T = 64, 128, 32, 8
HID, SP, HD, HE, HL, HLO = 64, 64, 2, 32, 48, 32
L = 2
TAU = 2.0
E = N * K
NC, NS, LN = 2, 16, 16
NW = NC * NS
BPW = B // NW
NB = N // LN
DC = HID // LN


EB = LN * K
NCH = HID // LN


def _sc_edges_body(tabflat, ai, posx, posy, gidx, idxl, maskl, wrow, acc,
                   rows_v, ai_v, px_v, py_v, gi_v, jl_v, mk_v, wr_v, acc_v,
                   d2_v, sem):
    i32 = jnp.int32
    wid = lax.axis_index("s") * NC + lax.axis_index("c")
    pltpu.sync_copy(wrow, wr_v)
    wr_c = [wr_v[pl.ds(c * LN, LN)] for c in range(NCH)]
    for bb in range(BPW):
        b = wid * BPW + bb
        pltpu.sync_copy(ai.at[b], ai_v)
        pltpu.sync_copy(posx.at[b], px_v)
        pltpu.sync_copy(posy.at[b], py_v)

        def blk_body(blk, carry_blk):
            pltpu.sync_copy(gidx.at[b, pl.ds(blk * 4, 4)], gi_v)
            pltpu.sync_copy(idxl.at[b, pl.ds(blk * EB, EB)], jl_v)
            pltpu.sync_copy(maskl.at[b, pl.ds(blk * EB, EB)], mk_v)
            cps = [pltpu.async_copy(tabflat.at[gi_v.at[j]],
                                    rows_v.at[pl.ds(j * 128, 128)], sem)
                   for j in range(4)]

            def pre_body(g, _):
                node = blk * LN + g // 2
                nsplat = jnp.full((LN,), node, i32)
                jv = jl_v[pl.ds(g * LN, LN)]
                gx = plsc.load_gather(px_v, [jv])
                gy = plsc.load_gather(py_v, [jv])
                pxi = plsc.load_gather(px_v, [nsplat])
                pyi = plsc.load_gather(py_v, [nsplat])
                rx = pxi - gx
                ry = pyi - gy
                d2_v[pl.ds(g * LN, LN)] = rx * rx + ry * ry
                return _

            plsc.parallel_loop(0, EB // LN, 1, unroll=4,
                               carry=jnp.int32(0))(pre_body)
            for cp in cps:
                cp.wait()

            for il in range(LN):
                node = blk * LN + il
                ai_c = [ai_v[node, pl.ds(c * LN, LN)] for c in range(NCH)]

                def k_body(k, carry):
                    e = il * K + k
                    esplat = jnp.full((LN,), e, i32)
                    sd2 = plsc.load_gather(d2_v, [esplat])
                    smv = plsc.load_gather(mk_v, [esplat])
                    out = []
                    for c in range(NCH):
                        row = rows_v[e, pl.ds(c * LN, LN)]
                        v = jnp.maximum(ai_c[c] + row + sd2 * wr_c[c], 0.0)
                        out.append(carry[c] + smv * v)
                    return tuple(out)

                acc0 = tuple(jnp.zeros((LN,), jnp.float32) for _ in range(NCH))
                accs = plsc.parallel_loop(0, K, 1, unroll=2, carry=acc0)(k_body)
                for c in range(NCH):
                    acc_v[node, pl.ds(c * LN, LN)] = accs[c]
            return carry_blk

        lax.fori_loop(0, NB, blk_body, 0)
        pltpu.sync_copy(acc_v, acc.at[b])


def _sc_edges(tabflat, ai, posx, posy, gidx, idxl, maskl, wrow):
    mesh = plsc.VectorSubcoreMesh(core_axis_name="c", subcore_axis_name="s")
    return pl.kernel(
        _sc_edges_body,
        mesh=mesh,
        compiler_params=pltpu.CompilerParams(needs_layout_passes=False,
                                             use_tc_tiling_on_sc=False),
        out_type=jax.ShapeDtypeStruct((B, N, HID), jnp.float32),
        scratch_types=[
            pltpu.VMEM((EB, HID), jnp.float32),
            pltpu.VMEM((N, HID), jnp.float32),
            pltpu.VMEM((N,), jnp.float32),
            pltpu.VMEM((N,), jnp.float32),
            pltpu.VMEM((4, 128), jnp.int32),
            pltpu.VMEM((EB,), jnp.int32),
            pltpu.VMEM((EB,), jnp.float32),
            pltpu.VMEM((HID,), jnp.float32),
            pltpu.VMEM((N, HID), jnp.float32),
            pltpu.VMEM((EB,), jnp.float32),
            pltpu.SemaphoreType.DMA,
        ],
    )(tabflat, ai, posx, posy, gidx, idxl, maskl, wrow)


def _a1_body(beta_ref, ped_ref, maskf_ref,
             Wef_ref, Wet_ref, bemb_ref, Wm1a0_ref, Wm1b0_ref, bm1r0_ref,
             h0_ref, ajt0_ref, ai0_ref, cnt_ref):
    f32 = jnp.float32
    beta = beta_ref[0, 0, 0]
    bvec = jnp.full((1, 1), beta, f32)
    sb = jnp.sin(bvec)
    cb = jnp.cos(bvec)
    trow = (beta * Wet_ref[0:1, :] + sb * Wet_ref[1:2, :]
            + cb * Wet_ref[2:3, :] + bemb_ref[...])
    ped = ped_ref[0]
    h = jnp.dot(ped, Wef_ref[...], preferred_element_type=f32) + trow
    h0_ref[0] = h
    ajt0_ref[0] = jnp.dot(h, Wm1b0_ref[...], preferred_element_type=f32)
    ai0_ref[0] = jnp.dot(h, Wm1a0_ref[...], preferred_element_type=f32) + bm1r0_ref[...]
    cnt_ref[0] = jnp.sum(maskf_ref[0], axis=1, keepdims=True)


def _a2_body(beta_ref, x_ref, self_ref, hist_ref,
             Whebd_ref, bhebd_ref, WxT_ref, WhT_ref, bg_ref, Wlo_ref, blo_ref,
             Wsp_ref, bsp_ref, Wc1b_ref, Wc1c_ref, Wc1t_ref, bc1_ref,
             part_ref, pred_ref):
    f32 = jnp.float32
    beta = beta_ref[0, 0, 0]
    bvec = jnp.full((1, 1), beta, f32)
    sb = jnp.sin(bvec)
    cb = jnp.cos(bvec)

    he = jnp.maximum(
        jnp.dot(hist_ref[0], Whebd_ref[...], preferred_element_type=f32)
        + bhebd_ref[...], 0.0)
    hs = jnp.zeros((N, HL), f32)
    cs = jnp.zeros((N, HL), f32)
    for tt in range(T):
        xt = he[:, HE * tt:HE * (tt + 1)]
        gates = (jnp.dot(xt, WxT_ref[...], preferred_element_type=f32)
                 + jnp.dot(hs, WhT_ref[...], preferred_element_type=f32)
                 + bg_ref[...])
        ig = jax.nn.sigmoid(gates[:, 0:HL])
        fg = jax.nn.sigmoid(gates[:, HL:2 * HL])
        gg = jnp.tanh(gates[:, 2 * HL:3 * HL])
        og = jax.nn.sigmoid(gates[:, 3 * HL:4 * HL])
        cs = fg * cs + ig * gg
        hs = og * jnp.tanh(cs)
    hist_out = jnp.dot(hs, Wlo_ref[...], preferred_element_type=f32) + blo_ref[...]

    spatial = jnp.maximum(
        jnp.dot(x_ref[0], Wsp_ref[...], preferred_element_type=f32)
        + bsp_ref[...], 0.0)
    sf = self_ref[0]
    ds = sf[:, 4:5]
    sx = sf[:, 0:1]
    sy = sf[:, 1:2]
    temp = jnp.sqrt(sx * sx + sy * sy)
    temp_ = jnp.where(temp == 0.0, temp + 0.1, temp)
    predx = (ds * sx / temp_ - sf[:, 2:3]) / TAU
    predy = (ds * sy / temp_ - sf[:, 3:4]) / TAU
    pred_ref[0] = jnp.concatenate([predx, predy], axis=1)

    trow2 = (beta * Wc1t_ref[0:1, :] + sb * Wc1t_ref[1:2, :]
             + cb * Wc1t_ref[2:3, :] + bc1_ref[...])
    part_ref[0] = (jnp.dot(hist_out, Wc1b_ref[...], preferred_element_type=f32)
                   + jnp.dot(spatial, Wc1c_ref[...], preferred_element_type=f32)
                   + trow2)


def _layer_update(h, acc, cnt, Wm2_ref, bm2_ref, Wu1a_ref, Wu1b_ref,
                  bu1_ref, Wu2_ref, bu2_ref):
    f32 = jnp.float32
    agg = (jnp.dot(acc, Wm2_ref[...], preferred_element_type=f32)
           + cnt * bm2_ref[...])
    upd = jnp.maximum(
        jnp.dot(h, Wu1a_ref[...], preferred_element_type=f32)
        + jnp.dot(agg, Wu1b_ref[...], preferred_element_type=f32)
        + bu1_ref[...], 0.0)
    return h + jnp.dot(upd, Wu2_ref[...], preferred_element_type=f32) + bu2_ref[...]


def _b_body(h0_ref, acc0_ref, cnt_ref,
            Wm20_ref, bm20_ref, Wu1a0_ref, Wu1b0_ref, bu10_ref, Wu20_ref,
            bu20_ref, Wm1a1_ref, Wm1b1_ref, bm1r1_ref,
            h1_ref, ajt1_ref, ai1_ref):
    f32 = jnp.float32
    h1 = _layer_update(h0_ref[0], acc0_ref[0], cnt_ref[0], Wm20_ref, bm20_ref,
                       Wu1a0_ref, Wu1b0_ref, bu10_ref, Wu20_ref, bu20_ref)
    h1_ref[0] = h1
    ajt1_ref[0] = jnp.dot(h1, Wm1b1_ref[...], preferred_element_type=f32)
    ai1_ref[0] = jnp.dot(h1, Wm1a1_ref[...], preferred_element_type=f32) + bm1r1_ref[...]


def _c_body(h1_ref, acc1_ref, cnt_ref, part_ref, pred_ref,
            Wm21_ref, bm21_ref, Wu1a1_ref, Wu1b1_ref, bu11_ref, Wu21_ref,
            bu21_ref, Wc1a_ref, Wd_ref, bd_ref, out_ref):
    f32 = jnp.float32
    h2 = _layer_update(h1_ref[0], acc1_ref[0], cnt_ref[0], Wm21_ref, bm21_ref,
                       Wu1a1_ref, Wu1b1_ref, bu11_ref, Wu21_ref, bu21_ref)
    hcat = jnp.dot(h2, Wc1a_ref[...], preferred_element_type=f32) + part_ref[0]
    out_ref[0] = (jnp.dot(jnp.maximum(hcat, 0.0), Wd_ref[...],
                          preferred_element_type=f32)
                  + bd_ref[...] + pred_ref[0])


def _bspec(shape):
    return pl.BlockSpec(shape, lambda b: (b,) + (0,) * (len(shape) - 1))


def _wspec(shape):
    return pl.BlockSpec(shape, lambda b: (0,) * len(shape))


_SMEM_BETA = pl.BlockSpec((1, 1, 1), lambda b: (b, 0, 0),
                          memory_space=pltpu.SMEM)


def _tc_call(body, in_specs, out_specs, out_shapes, args):
    return pl.pallas_call(
        body, grid=(B,), in_specs=in_specs, out_specs=out_specs,
        out_shape=out_shapes)(*args)


@jax.jit
def _run(beta2, x, ped, maskf, selff, posx, posy, gidx, idxl, maskl, hist2,
         Wef, Wet, bemb,
         Wm1a, Wm1b, bm1r, wrow, Wm2, bm2r, Wu1a, Wu1b, bu1r, Wu2, bu2r,
         Whebd, bhebd, WxT, WhT, bg, Wlo, blo, Wsp, bsp,
         Wc1a, Wc1b, Wc1c, Wc1t, bc1, Wd, bd):
    f32 = jnp.float32
    W = lambda s: _wspec(s)

    h0, ajt0, ai0, cnt = _tc_call(
        _a1_body,
        [_SMEM_BETA, _bspec((1, N, 6)), _bspec((1, N, K)),
         W((6, HID)), W((3, HID)), W((1, HID)),
         W((HID, HID)), W((HID, HID)), W((1, HID))],
        [_bspec((1, N, HID)), _bspec((1, N, HID)), _bspec((1, N, HID)),
         _bspec((1, N, 1))],
        [jax.ShapeDtypeStruct((B, N, HID), f32),
         jax.ShapeDtypeStruct((B, N, HID), f32),
         jax.ShapeDtypeStruct((B, N, HID), f32),
         jax.ShapeDtypeStruct((B, N, 1), f32)],
        [beta2, ped, maskf, Wef, Wet, bemb, Wm1a[0], Wm1b[0], bm1r[0]])

    acc0 = _sc_edges(ajt0.reshape(B * N, HID), ai0, posx, posy,
                     gidx, idxl, maskl, wrow[0])

    part, pred = _tc_call(
        _a2_body,
        [_SMEM_BETA, _bspec((1, N, 2)), _bspec((1, N, 5)),
         _bspec((1, N, T * HD)),
         W((T * HD, T * HE)), W((1, T * HE)),
         W((HE, 4 * HL)), W((HL, 4 * HL)), W((1, 4 * HL)),
         W((HL, HLO)), W((1, HLO)), W((2, SP)), W((1, SP)),
         W((HLO, SP // 2)), W((SP, SP // 2)), W((3, SP // 2)),
         W((1, SP // 2))],
        [_bspec((1, N, SP // 2)), _bspec((1, N, 2))],
        [jax.ShapeDtypeStruct((B, N, SP // 2), f32),
         jax.ShapeDtypeStruct((B, N, 2), f32)],
        [beta2, x, selff, hist2, Whebd, bhebd, WxT, WhT, bg, Wlo, blo,
         Wsp, bsp, Wc1b, Wc1c, Wc1t, bc1])

    h1, ajt1, ai1 = _tc_call(
        _b_body,
        [_bspec((1, N, HID)), _bspec((1, N, HID)), _bspec((1, N, 1)),
         W((HID, HID)), W((1, HID)), W((HID, HID)), W((HID, HID)),
         W((1, HID)), W((HID, HID)), W((1, HID)),
         W((HID, HID)), W((HID, HID)), W((1, HID))],
        [_bspec((1, N, HID)), _bspec((1, N, HID)), _bspec((1, N, HID))],
        [jax.ShapeDtypeStruct((B, N, HID), f32),
         jax.ShapeDtypeStruct((B, N, HID), f32),
         jax.ShapeDtypeStruct((B, N, HID), f32)],
        [h0, acc0, cnt, Wm2[0], bm2r[0], Wu1a[0], Wu1b[0], bu1r[0],
         Wu2[0], bu2r[0], Wm1a[1], Wm1b[1], bm1r[1]])

    acc1 = _sc_edges(ajt1.reshape(B * N, HID), ai1, posx, posy,
                     gidx, idxl, maskl, wrow[1])

    out = _tc_call(
        _c_body,
        [_bspec((1, N, HID)), _bspec((1, N, HID)), _bspec((1, N, 1)),
         _bspec((1, N, SP // 2)), _bspec((1, N, 2)),
         W((HID, HID)), W((1, HID)), W((HID, HID)), W((HID, HID)),
         W((1, HID)), W((HID, HID)), W((1, HID)),
         W((HID, SP // 2)), W((SP // 2, 2)), W((1, 2))],
        [_bspec((1, N, 2))],
        [jax.ShapeDtypeStruct((B, N, 2), f32)],
        [h1, acc1, cnt, part, pred, Wm2[1], bm2r[1], Wu1a[1], Wu1b[1],
         bu1r[1], Wu2[1], bu2r[1], Wc1a, Wd, bd])
    return out[0] if isinstance(out, (list, tuple)) else out


def kernel(x, beta, ped_features, neigh_ped_mask, self_features, near_ped_idx,
           hist_feature, nei_list, t, W_emb, b_emb, Wm1, bm1, Wm2, bm2,
           Wu1, bu1, Wu2, bu2, W_sp, b_sp, W_he, b_he, Wih, Whh, bih, bhh,
           W_lo, b_lo, W_c1, b_c1, W_d, b_d):
    f32 = jnp.float32
    beta2 = beta.reshape(B, 1, 1).astype(f32)
    maskf = neigh_ped_mask.astype(f32)
    idx = near_ped_idx.astype(jnp.int32)
    idxl = idx.reshape(B, E)
    gidx = (idx + (jnp.arange(B, dtype=jnp.int32) * N)[:, None, None]
            ).reshape(B, E // 128, 128)
    maskl = maskf.reshape(B, E)
    posx = ped_features[:, :, 0].astype(f32)
    posy = ped_features[:, :, 1].astype(f32)
    hist2 = hist_feature.reshape(B, N, T * HD).astype(f32)

    Wef = W_emb[:6].astype(f32)
    Wet = W_emb[6:9].astype(f32)
    bemb = b_emb.reshape(1, HID).astype(f32)
    Wm1a = Wm1[:, :HID, :].astype(f32)
    Wm1b = Wm1[:, HID:2 * HID, :].astype(f32)
    bm1r = bm1.reshape(L, 1, HID).astype(f32)
    wrow = Wm1[:, 2 * HID, :].astype(f32)
    Wm2c = Wm2.astype(f32)
    bm2r = bm2.reshape(L, 1, HID).astype(f32)
    Wu1a = Wu1[:, :HID, :].astype(f32)
    Wu1b = Wu1[:, HID:, :].astype(f32)
    bu1r = bu1.reshape(L, 1, HID).astype(f32)
    Wu2c = Wu2.astype(f32)
    bu2r = bu2.reshape(L, 1, HID).astype(f32)
    Whebd = jnp.zeros((T * HD, T * HE), f32)
    for tt in range(T):
        Whebd = Whebd.at[tt * HD:(tt + 1) * HD, tt * HE:(tt + 1) * HE].set(
            W_he.astype(f32))
    bhebd = jnp.tile(b_he.astype(f32), T).reshape(1, T * HE)
    WxT = Wih.T.astype(f32)
    WhT = Whh.T.astype(f32)
    bg = (bih + bhh).reshape(1, 4 * HL).astype(f32)
    Wlo = W_lo.astype(f32)
    blo = b_lo.reshape(1, HLO).astype(f32)
    Wsp = W_sp.astype(f32)
    bsp = b_sp.reshape(1, SP).astype(f32)
    Wc1a = W_c1[0:HID].astype(f32)
    Wc1b = W_c1[HID:HID + HLO].astype(f32)
    Wc1c = W_c1[HID + HLO:HID + HLO + SP].astype(f32)
    Wc1t = W_c1[HID + HLO + SP:].astype(f32)
    bc1 = b_c1.reshape(1, SP // 2).astype(f32)
    Wd = W_d.astype(f32)
    bd = b_d.reshape(1, 2).astype(f32)

    return _run(beta2, x.astype(f32), ped_features.astype(f32), maskf,
                self_features.astype(f32), posx, posy, gidx, idxl, maskl,
                hist2, Wef, Wet, bemb,
                Wm1a, Wm1b, bm1r, wrow, Wm2c, bm2r, Wu1a, Wu1b, bu1r,
                Wu2c, bu2r,
                Whebd, bhebd, WxT, WhT, bg, Wlo, blo, Wsp, bsp,
                Wc1a, Wc1b, Wc1c, Wc1t, bc1, Wd, bd)


# speedup vs baseline: 2.2971x; 1.7667x over previous
import functools
import jax
import jax.numpy as jnp
from jax import lax
from jax.experimental import pallas as pl
from jax.experimental.pallas import tpu as pltpu
from jax.experimental.pallas import tpu_sc as plsc

B, N, K, T = 64, 128, 32, 8
HID, SP, HD, HE, HL, HLO = 64, 64, 2, 32, 48, 32
L = 2
TAU = 2.0
E = N * K
NC, NS, LN = 2, 16, 16
NW = NC * NS
BPW = B // NW
NB = N // LN
DC = HID // LN


EB = LN * K
NCH = HID // LN


def _sc_edges_body(tabflat, ai, posx, posy, gidx, idxl, maskl, wrow, acc,
                   rows_v, ai_v, px_v, py_v, gi_v, jl_v, mk_v, wr_v, acc_v,
                   d2_v, sem):
    i32 = jnp.int32
    wid = lax.axis_index("s") * NC + lax.axis_index("c")
    pltpu.sync_copy(wrow, wr_v)
    wr_c = [wr_v[pl.ds(c * LN, LN)] for c in range(NCH)]
    for bb in range(BPW):
        b = wid * BPW + bb
        pltpu.sync_copy(ai.at[b], ai_v)
        pltpu.sync_copy(posx.at[b], px_v)
        pltpu.sync_copy(posy.at[b], py_v)

        def blk_body(blk, carry_blk):
            pltpu.sync_copy(gidx.at[b, pl.ds(blk * 4, 4)], gi_v)
            pltpu.sync_copy(idxl.at[b, pl.ds(blk * EB, EB)], jl_v)
            pltpu.sync_copy(maskl.at[b, pl.ds(blk * EB, EB)], mk_v)
            cps = [pltpu.async_copy(tabflat.at[gi_v.at[j]],
                                    rows_v.at[pl.ds(j * 128, 128)], sem)
                   for j in range(4)]

            def pre_body(g, _):
                node = blk * LN + g // 2
                nsplat = jnp.full((LN,), node, i32)
                jv = jl_v[pl.ds(g * LN, LN)]
                gx = plsc.load_gather(px_v, [jv])
                gy = plsc.load_gather(py_v, [jv])
                pxi = plsc.load_gather(px_v, [nsplat])
                pyi = plsc.load_gather(py_v, [nsplat])
                rx = pxi - gx
                ry = pyi - gy
                d2_v[pl.ds(g * LN, LN)] = rx * rx + ry * ry
                return _

            plsc.parallel_loop(0, EB // LN, 1, unroll=4,
                               carry=jnp.int32(0))(pre_body)
            for cp in cps:
                cp.wait()

            for il in range(LN):
                node = blk * LN + il
                ai_c = [ai_v[node, pl.ds(c * LN, LN)] for c in range(NCH)]

                def k_body(k, carry):
                    e = il * K + k
                    esplat = jnp.full((LN,), e, i32)
                    sd2 = plsc.load_gather(d2_v, [esplat])
                    smv = plsc.load_gather(mk_v, [esplat])
                    out = []
                    for c in range(NCH):
                        row = rows_v[e, pl.ds(c * LN, LN)]
                        v = jnp.maximum(ai_c[c] + row + sd2 * wr_c[c], 0.0)
                        out.append(carry[c] + smv * v)
                    return tuple(out)

                acc0 = tuple(jnp.zeros((LN,), jnp.float32) for _ in range(NCH))
                accs = plsc.parallel_loop(0, K, 1, unroll=2, carry=acc0)(k_body)
                for c in range(NCH):
                    acc_v[node, pl.ds(c * LN, LN)] = accs[c]
            return carry_blk

        lax.fori_loop(0, NB, blk_body, 0)
        pltpu.sync_copy(acc_v, acc.at[b])


def _sc_edges(tabflat, ai, posx, posy, gidx, idxl, maskl, wrow):
    mesh = plsc.VectorSubcoreMesh(core_axis_name="c", subcore_axis_name="s")
    return pl.kernel(
        _sc_edges_body,
        mesh=mesh,
        compiler_params=pltpu.CompilerParams(needs_layout_passes=False,
                                             use_tc_tiling_on_sc=False),
        out_type=jax.ShapeDtypeStruct((B, N, HID), jnp.float32),
        scratch_types=[
            pltpu.VMEM((EB, HID), jnp.float32),
            pltpu.VMEM((N, HID), jnp.float32),
            pltpu.VMEM((N,), jnp.float32),
            pltpu.VMEM((N,), jnp.float32),
            pltpu.VMEM((4, 128), jnp.int32),
            pltpu.VMEM((EB,), jnp.int32),
            pltpu.VMEM((EB,), jnp.float32),
            pltpu.VMEM((HID,), jnp.float32),
            pltpu.VMEM((N, HID), jnp.float32),
            pltpu.VMEM((EB,), jnp.float32),
            pltpu.SemaphoreType.DMA,
        ],
    )(tabflat, ai, posx, posy, gidx, idxl, maskl, wrow)


GB = 8
GR = GB * N


def _trow3(beta_v, W3, brow):
    t = (beta_v * W3[0:1, :] + jnp.sin(beta_v) * W3[1:2, :]
         + jnp.cos(beta_v) * W3[2:3, :] + brow)
    H = t.shape[1]
    return jnp.broadcast_to(t.reshape(GB, 1, H), (GB, N, H)).reshape(GR, H)


def _a1_body(beta_ref, ped_ref, maskf_ref,
             Wef_ref, Wet_ref, bemb_ref, Wm1a0_ref, Wm1b0_ref, bm1r0_ref,
             h0_ref, ajt0_ref, ai0_ref, cnt_ref):
    f32 = jnp.float32
    trow = _trow3(beta_ref[...], Wet_ref[...], bemb_ref[...])
    ped = ped_ref[...].reshape(GR, 6)
    h = jnp.dot(ped, Wef_ref[...], preferred_element_type=f32) + trow
    h0_ref[...] = h.reshape(GB, N, HID)
    ajt0_ref[...] = jnp.dot(h, Wm1b0_ref[...],
                            preferred_element_type=f32).reshape(GB, N, HID)
    ai0_ref[...] = (jnp.dot(h, Wm1a0_ref[...], preferred_element_type=f32)
                    + bm1r0_ref[...]).reshape(GB, N, HID)
    cnt_ref[...] = jnp.sum(maskf_ref[...], axis=2, keepdims=True)


def _a2_body(beta_ref, x_ref, self_ref, hist_ref,
             Whebd_ref, bhebd_ref, WxT_ref, WhT_ref, bg_ref, Wlo_ref, blo_ref,
             Wsp_ref, bsp_ref, Wc1b_ref, Wc1c_ref, Wc1t_ref, bc1_ref,
             part_ref, pred_ref):
    f32 = jnp.float32
    he = jnp.maximum(
        jnp.dot(hist_ref[...].reshape(GR, T * HD), Whebd_ref[...],
                preferred_element_type=f32)
        + bhebd_ref[...], 0.0)
    hs = jnp.zeros((GR, HL), f32)
    cs = jnp.zeros((GR, HL), f32)
    for tt in range(T):
        xt = he[:, HE * tt:HE * (tt + 1)]
        gates = (jnp.dot(xt, WxT_ref[...], preferred_element_type=f32)
                 + jnp.dot(hs, WhT_ref[...], preferred_element_type=f32)
                 + bg_ref[...])
        ig = jax.nn.sigmoid(gates[:, 0:HL])
        fg = jax.nn.sigmoid(gates[:, HL:2 * HL])
        gg = jnp.tanh(gates[:, 2 * HL:3 * HL])
        og = jax.nn.sigmoid(gates[:, 3 * HL:4 * HL])
        cs = fg * cs + ig * gg
        hs = og * jnp.tanh(cs)
    hist_out = jnp.dot(hs, Wlo_ref[...], preferred_element_type=f32) + blo_ref[...]

    spatial = jnp.maximum(
        jnp.dot(x_ref[...].reshape(GR, 2), Wsp_ref[...],
                preferred_element_type=f32)
        + bsp_ref[...], 0.0)
    sf = self_ref[...].reshape(GR, 5)
    ds = sf[:, 4:5]
    sx = sf[:, 0:1]
    sy = sf[:, 1:2]
    temp = jnp.sqrt(sx * sx + sy * sy)
    temp_ = jnp.where(temp == 0.0, temp + 0.1, temp)
    predx = (ds * sx / temp_ - sf[:, 2:3]) / TAU
    predy = (ds * sy / temp_ - sf[:, 3:4]) / TAU
    pred_ref[...] = jnp.concatenate([predx, predy], axis=1).reshape(GB, N, 2)

    trow2 = _trow3(beta_ref[...], Wc1t_ref[...], bc1_ref[...])
    part_ref[...] = (jnp.dot(hist_out, Wc1b_ref[...], preferred_element_type=f32)
                     + jnp.dot(spatial, Wc1c_ref[...], preferred_element_type=f32)
                     + trow2).reshape(GB, N, SP // 2)


def _layer_update(h, acc, cnt, Wm2_ref, bm2_ref, Wu1a_ref, Wu1b_ref,
                  bu1_ref, Wu2_ref, bu2_ref):
    f32 = jnp.float32
    agg = (jnp.dot(acc, Wm2_ref[...], preferred_element_type=f32)
           + cnt * bm2_ref[...])
    upd = jnp.maximum(
        jnp.dot(h, Wu1a_ref[...], preferred_element_type=f32)
        + jnp.dot(agg, Wu1b_ref[...], preferred_element_type=f32)
        + bu1_ref[...], 0.0)
    return h + jnp.dot(upd, Wu2_ref[...], preferred_element_type=f32) + bu2_ref[...]


def _b_body(h0_ref, acc0_ref, cnt_ref,
            Wm20_ref, bm20_ref, Wu1a0_ref, Wu1b0_ref, bu10_ref, Wu20_ref,
            bu20_ref, Wm1a1_ref, Wm1b1_ref, bm1r1_ref,
            h1_ref, ajt1_ref, ai1_ref):
    f32 = jnp.float32
    h1 = _layer_update(h0_ref[...].reshape(GR, HID),
                       acc0_ref[...].reshape(GR, HID),
                       cnt_ref[...].reshape(GR, 1), Wm20_ref, bm20_ref,
                       Wu1a0_ref, Wu1b0_ref, bu10_ref, Wu20_ref, bu20_ref)
    h1_ref[...] = h1.reshape(GB, N, HID)
    ajt1_ref[...] = jnp.dot(h1, Wm1b1_ref[...],
                            preferred_element_type=f32).reshape(GB, N, HID)
    ai1_ref[...] = (jnp.dot(h1, Wm1a1_ref[...], preferred_element_type=f32)
                    + bm1r1_ref[...]).reshape(GB, N, HID)


def _c_body(h1_ref, acc1_ref, cnt_ref, part_ref, pred_ref,
            Wm21_ref, bm21_ref, Wu1a1_ref, Wu1b1_ref, bu11_ref, Wu21_ref,
            bu21_ref, Wc1a_ref, Wd_ref, bd_ref, out_ref):
    f32 = jnp.float32
    h2 = _layer_update(h1_ref[...].reshape(GR, HID),
                       acc1_ref[...].reshape(GR, HID),
                       cnt_ref[...].reshape(GR, 1), Wm21_ref, bm21_ref,
                       Wu1a1_ref, Wu1b1_ref, bu11_ref, Wu21_ref, bu21_ref)
    hcat = (jnp.dot(h2, Wc1a_ref[...], preferred_element_type=f32)
            + part_ref[...].reshape(GR, SP // 2))
    out_ref[...] = (jnp.dot(jnp.maximum(hcat, 0.0), Wd_ref[...],
                            preferred_element_type=f32)
                    + bd_ref[...]
                    + pred_ref[...].reshape(GR, 2)).reshape(GB, N, 2)


def _bspec(shape):
    return pl.BlockSpec(shape, lambda b: (b,) + (0,) * (len(shape) - 1))


def _wspec(shape):
    return pl.BlockSpec(shape, lambda b: (0,) * len(shape))


_BETA_SPEC = pl.BlockSpec((GB, 1), lambda b: (b, 0))


def _tc_call(body, in_specs, out_specs, out_shapes, args):
    return pl.pallas_call(
        body, grid=(B // GB,), in_specs=in_specs, out_specs=out_specs,
        out_shape=out_shapes)(*args)


@jax.jit
def _run(beta2, x, ped, maskf, selff, posx, posy, gidx, idxl, maskl, hist2,
         Wef, Wet, bemb,
         Wm1a, Wm1b, bm1r, wrow, Wm2, bm2r, Wu1a, Wu1b, bu1r, Wu2, bu2r,
         Whebd, bhebd, WxT, WhT, bg, Wlo, blo, Wsp, bsp,
         Wc1a, Wc1b, Wc1c, Wc1t, bc1, Wd, bd):
    f32 = jnp.float32
    W = lambda s: _wspec(s)

    h0, ajt0, ai0, cnt = _tc_call(
        _a1_body,
        [_BETA_SPEC, _bspec((GB, N, 6)), _bspec((GB, N, K)),
         W((6, HID)), W((3, HID)), W((1, HID)),
         W((HID, HID)), W((HID, HID)), W((1, HID))],
        [_bspec((GB, N, HID)), _bspec((GB, N, HID)), _bspec((GB, N, HID)),
         _bspec((GB, N, 1))],
        [jax.ShapeDtypeStruct((B, N, HID), f32),
         jax.ShapeDtypeStruct((B, N, HID), f32),
         jax.ShapeDtypeStruct((B, N, HID), f32),
         jax.ShapeDtypeStruct((B, N, 1), f32)],
        [beta2, ped, maskf, Wef, Wet, bemb, Wm1a[0], Wm1b[0], bm1r[0]])

    acc0 = _sc_edges(ajt0.reshape(B * N, HID), ai0, posx, posy,
                     gidx, idxl, maskl, wrow[0])

    part, pred = _tc_call(
        _a2_body,
        [_BETA_SPEC, _bspec((GB, N, 2)), _bspec((GB, N, 5)),
         _bspec((GB, N, T * HD)),
         W((T * HD, T * HE)), W((1, T * HE)),
         W((HE, 4 * HL)), W((HL, 4 * HL)), W((1, 4 * HL)),
         W((HL, HLO)), W((1, HLO)), W((2, SP)), W((1, SP)),
         W((HLO, SP // 2)), W((SP, SP // 2)), W((3, SP // 2)),
         W((1, SP // 2))],
        [_bspec((GB, N, SP // 2)), _bspec((GB, N, 2))],
        [jax.ShapeDtypeStruct((B, N, SP // 2), f32),
         jax.ShapeDtypeStruct((B, N, 2), f32)],
        [beta2, x, selff, hist2, Whebd, bhebd, WxT, WhT, bg, Wlo, blo,
         Wsp, bsp, Wc1b, Wc1c, Wc1t, bc1])

    h1, ajt1, ai1 = _tc_call(
        _b_body,
        [_bspec((GB, N, HID)), _bspec((GB, N, HID)), _bspec((GB, N, 1)),
         W((HID, HID)), W((1, HID)), W((HID, HID)), W((HID, HID)),
         W((1, HID)), W((HID, HID)), W((1, HID)),
         W((HID, HID)), W((HID, HID)), W((1, HID))],
        [_bspec((GB, N, HID)), _bspec((GB, N, HID)), _bspec((GB, N, HID))],
        [jax.ShapeDtypeStruct((B, N, HID), f32),
         jax.ShapeDtypeStruct((B, N, HID), f32),
         jax.ShapeDtypeStruct((B, N, HID), f32)],
        [h0, acc0, cnt, Wm2[0], bm2r[0], Wu1a[0], Wu1b[0], bu1r[0],
         Wu2[0], bu2r[0], Wm1a[1], Wm1b[1], bm1r[1]])

    acc1 = _sc_edges(ajt1.reshape(B * N, HID), ai1, posx, posy,
                     gidx, idxl, maskl, wrow[1])

    out = _tc_call(
        _c_body,
        [_bspec((GB, N, HID)), _bspec((GB, N, HID)), _bspec((GB, N, 1)),
         _bspec((GB, N, SP // 2)), _bspec((GB, N, 2)),
         W((HID, HID)), W((1, HID)), W((HID, HID)), W((HID, HID)),
         W((1, HID)), W((HID, HID)), W((1, HID)),
         W((HID, SP // 2)), W((SP // 2, 2)), W((1, 2))],
        [_bspec((GB, N, 2))],
        [jax.ShapeDtypeStruct((B, N, 2), f32)],
        [h1, acc1, cnt, part, pred, Wm2[1], bm2r[1], Wu1a[1], Wu1b[1],
         bu1r[1], Wu2[1], bu2r[1], Wc1a, Wd, bd])
    return out[0] if isinstance(out, (list, tuple)) else out


def kernel(x, beta, ped_features, neigh_ped_mask, self_features, near_ped_idx,
           hist_feature, nei_list, t, W_emb, b_emb, Wm1, bm1, Wm2, bm2,
           Wu1, bu1, Wu2, bu2, W_sp, b_sp, W_he, b_he, Wih, Whh, bih, bhh,
           W_lo, b_lo, W_c1, b_c1, W_d, b_d):
    f32 = jnp.float32
    beta2 = beta.reshape(B, 1).astype(f32)
    maskf = neigh_ped_mask.astype(f32)
    idx = near_ped_idx.astype(jnp.int32)
    idxl = idx.reshape(B, E)
    gidx = (idx + (jnp.arange(B, dtype=jnp.int32) * N)[:, None, None]
            ).reshape(B, E // 128, 128)
    maskl = maskf.reshape(B, E)
    posx = ped_features[:, :, 0].astype(f32)
    posy = ped_features[:, :, 1].astype(f32)
    hist2 = hist_feature.reshape(B, N, T * HD).astype(f32)

    Wef = W_emb[:6].astype(f32)
    Wet = W_emb[6:9].astype(f32)
    bemb = b_emb.reshape(1, HID).astype(f32)
    Wm1a = Wm1[:, :HID, :].astype(f32)
    Wm1b = Wm1[:, HID:2 * HID, :].astype(f32)
    bm1r = bm1.reshape(L, 1, HID).astype(f32)
    wrow = Wm1[:, 2 * HID, :].astype(f32)
    Wm2c = Wm2.astype(f32)
    bm2r = bm2.reshape(L, 1, HID).astype(f32)
    Wu1a = Wu1[:, :HID, :].astype(f32)
    Wu1b = Wu1[:, HID:, :].astype(f32)
    bu1r = bu1.reshape(L, 1, HID).astype(f32)
    Wu2c = Wu2.astype(f32)
    bu2r = bu2.reshape(L, 1, HID).astype(f32)
    Whebd = jnp.zeros((T * HD, T * HE), f32)
    for tt in range(T):
        Whebd = Whebd.at[tt * HD:(tt + 1) * HD, tt * HE:(tt + 1) * HE].set(
            W_he.astype(f32))
    bhebd = jnp.tile(b_he.astype(f32), T).reshape(1, T * HE)
    WxT = Wih.T.astype(f32)
    WhT = Whh.T.astype(f32)
    bg = (bih + bhh).reshape(1, 4 * HL).astype(f32)
    Wlo = W_lo.astype(f32)
    blo = b_lo.reshape(1, HLO).astype(f32)
    Wsp = W_sp.astype(f32)
    bsp = b_sp.reshape(1, SP).astype(f32)
    Wc1a = W_c1[0:HID].astype(f32)
    Wc1b = W_c1[HID:HID + HLO].astype(f32)
    Wc1c = W_c1[HID + HLO:HID + HLO + SP].astype(f32)
    Wc1t = W_c1[HID + HLO + SP:].astype(f32)
    bc1 = b_c1.reshape(1, SP // 2).astype(f32)
    Wd = W_d.astype(f32)
    bd = b_d.reshape(1, 2).astype(f32)

    return _run(beta2, x.astype(f32), ped_features.astype(f32), maskf,
                self_features.astype(f32), posx, posy, gidx, idxl, maskl,
                hist2, Wef, Wet, bemb,
                Wm1a, Wm1b, bm1r, wrow, Wm2c, bm2r, Wu1a, Wu1b, bu1r,
                Wu2c, bu2r,
                Whebd, bhebd, WxT, WhT, bg, Wlo, blo, Wsp, bsp,
                Wc1a, Wc1b, Wc1c, Wc1t, bc1, Wd, bd)


# speedup vs baseline: 2.6052x; 1.1341x over previous
import functools
import jax
import jax.numpy as jnp
from jax import lax
from jax.experimental import pallas as pl
from jax.experimental.pallas import tpu as pltpu
from jax.experimental.pallas import tpu_sc as plsc

B, N, K, T = 64, 128, 32, 8
HID, SP, HD, HE, HL, HLO = 64, 64, 2, 32, 48, 32
L = 2
TAU = 2.0
E = N * K
NC, NS, LN = 2, 16, 16
NW = NC * NS
BPW = B // NW
NB = N // LN
DC = HID // LN


EB = LN * K
NCH = HID // LN


def _sc_edges_body(tabflat, ai, posx, posy, gidx, idxl, maskl, wrow, acc,
                   rows0_v, rows1_v, ai_v, px_v, py_v, gi0_v, gi1_v, jl_v,
                   mk_v, wr_v, acc_v, d2_v, sem0, sem1):
    i32 = jnp.int32
    wid = lax.axis_index("s") * NC + lax.axis_index("c")
    pltpu.sync_copy(wrow, wr_v)
    wr_c = [wr_v[pl.ds(c * LN, LN)] for c in range(NCH)]
    bufs = ((rows0_v, gi0_v, sem0), (rows1_v, gi1_v, sem1))

    def fire(blk, par):
        rows, gi, sem = bufs[par]
        return [pltpu.async_copy(tabflat.at[gi.at[j]],
                                 rows.at[pl.ds(j * 128, 128)], sem)
                for j in range(4)]

    for bb in range(BPW):
        b = wid * BPW + bb
        pltpu.sync_copy(ai.at[b], ai_v)
        pltpu.sync_copy(posx.at[b], px_v)
        pltpu.sync_copy(posy.at[b], py_v)
        pltpu.sync_copy(gidx.at[b, pl.ds(0, 4)], gi0_v)
        fire(0, 0)

        def pair_body(bi, carry_pair):
            for par in range(2):
                blk = bi * 2 + par
                rows, gi_cur, sem = bufs[par]
                nrows, ngi, nsem = bufs[1 - par]
                pltpu.sync_copy(idxl.at[b, pl.ds(blk * EB, EB)], jl_v)
                pltpu.sync_copy(maskl.at[b, pl.ds(blk * EB, EB)], mk_v)

                @pl.when(blk + 1 < NB)
                def _prefetch():
                    pltpu.sync_copy(gidx.at[b, pl.ds((blk + 1) * 4, 4)], ngi)
                    fire(blk + 1, 1 - par)

                def pre_body(g, _):
                    node = blk * LN + g // 2
                    nsplat = jnp.full((LN,), node, i32)
                    jv = jl_v[pl.ds(g * LN, LN)]
                    gx = plsc.load_gather(px_v, [jv])
                    gy = plsc.load_gather(py_v, [jv])
                    pxi = plsc.load_gather(px_v, [nsplat])
                    pyi = plsc.load_gather(py_v, [nsplat])
                    rx = pxi - gx
                    ry = pyi - gy
                    d2_v[pl.ds(g * LN, LN)] = rx * rx + ry * ry
                    return _

                plsc.parallel_loop(0, EB // LN, 1, unroll=4,
                                   carry=jnp.int32(0))(pre_body)
                for j in range(4):
                    pltpu.make_async_copy(
                        tabflat.at[gi_cur.at[j]],
                        rows.at[pl.ds(j * 128, 128)], sem).wait()

                for il in range(LN):
                    node = blk * LN + il
                    ai_c = [ai_v[node, pl.ds(c * LN, LN)] for c in range(NCH)]

                    def k_body(k, carry):
                        e = il * K + k
                        esplat = jnp.full((LN,), e, i32)
                        sd2 = plsc.load_gather(d2_v, [esplat])
                        smv = plsc.load_gather(mk_v, [esplat])
                        out = []
                        for c in range(NCH):
                            row = rows[e, pl.ds(c * LN, LN)]
                            v = jnp.maximum(ai_c[c] + row + sd2 * wr_c[c], 0.0)
                            out.append(carry[c] + smv * v)
                        return tuple(out)

                    acc0 = tuple(jnp.zeros((LN,), jnp.float32)
                                 for _ in range(NCH))
                    accs = plsc.parallel_loop(0, K, 1, unroll=2,
                                              carry=acc0)(k_body)
                    for c in range(NCH):
                        acc_v[node, pl.ds(c * LN, LN)] = accs[c]
            return carry_pair

        lax.fori_loop(0, NB // 2, pair_body, 0)
        pltpu.sync_copy(acc_v, acc.at[b])


def _sc_edges(tabflat, ai, posx, posy, gidx, idxl, maskl, wrow):
    mesh = plsc.VectorSubcoreMesh(core_axis_name="c", subcore_axis_name="s")
    return pl.kernel(
        _sc_edges_body,
        mesh=mesh,
        compiler_params=pltpu.CompilerParams(needs_layout_passes=False,
                                             use_tc_tiling_on_sc=False),
        out_type=jax.ShapeDtypeStruct((B, N, HID), jnp.float32),
        scratch_types=[
            pltpu.VMEM((EB, HID), jnp.float32),
            pltpu.VMEM((EB, HID), jnp.float32),
            pltpu.VMEM((N, HID), jnp.float32),
            pltpu.VMEM((N,), jnp.float32),
            pltpu.VMEM((N,), jnp.float32),
            pltpu.VMEM((4, 128), jnp.int32),
            pltpu.VMEM((4, 128), jnp.int32),
            pltpu.VMEM((EB,), jnp.int32),
            pltpu.VMEM((EB,), jnp.float32),
            pltpu.VMEM((HID,), jnp.float32),
            pltpu.VMEM((N, HID), jnp.float32),
            pltpu.VMEM((EB,), jnp.float32),
            pltpu.SemaphoreType.DMA,
            pltpu.SemaphoreType.DMA,
        ],
    )(tabflat, ai, posx, posy, gidx, idxl, maskl, wrow)


GB = 8
GR = GB * N


def _trow3(beta_v, W3, brow):
    t = (beta_v * W3[0:1, :] + jnp.sin(beta_v) * W3[1:2, :]
         + jnp.cos(beta_v) * W3[2:3, :] + brow)
    H = t.shape[1]
    return jnp.broadcast_to(t.reshape(GB, 1, H), (GB, N, H)).reshape(GR, H)


def _a1_body(beta_ref, ped_ref, maskf_ref,
             Wef_ref, Wet_ref, bemb_ref, Wm1a0_ref, Wm1b0_ref, bm1r0_ref,
             h0_ref, ajt0_ref, ai0_ref, cnt_ref):
    f32 = jnp.float32
    trow = _trow3(beta_ref[...], Wet_ref[...], bemb_ref[...])
    ped = ped_ref[...].reshape(GR, 6)
    h = jnp.dot(ped, Wef_ref[...], preferred_element_type=f32) + trow
    h0_ref[...] = h.reshape(GB, N, HID)
    ajt0_ref[...] = jnp.dot(h, Wm1b0_ref[...],
                            preferred_element_type=f32).reshape(GB, N, HID)
    ai0_ref[...] = (jnp.dot(h, Wm1a0_ref[...], preferred_element_type=f32)
                    + bm1r0_ref[...]).reshape(GB, N, HID)
    cnt_ref[...] = jnp.sum(maskf_ref[...], axis=2, keepdims=True)


def _a2_body(beta_ref, x_ref, self_ref, hist_ref,
             Whebd_ref, bhebd_ref, WxT_ref, WhT_ref, bg_ref, Wlo_ref, blo_ref,
             Wsp_ref, bsp_ref, Wc1b_ref, Wc1c_ref, Wc1t_ref, bc1_ref,
             part_ref, pred_ref):
    f32 = jnp.float32
    he = jnp.maximum(
        jnp.dot(hist_ref[...].reshape(GR, T * HD), Whebd_ref[...],
                preferred_element_type=f32)
        + bhebd_ref[...], 0.0)
    hs = jnp.zeros((GR, HL), f32)
    cs = jnp.zeros((GR, HL), f32)
    for tt in range(T):
        xt = he[:, HE * tt:HE * (tt + 1)]
        gates = (jnp.dot(xt, WxT_ref[...], preferred_element_type=f32)
                 + jnp.dot(hs, WhT_ref[...], preferred_element_type=f32)
                 + bg_ref[...])
        ig = jax.nn.sigmoid(gates[:, 0:HL])
        fg = jax.nn.sigmoid(gates[:, HL:2 * HL])
        gg = jnp.tanh(gates[:, 2 * HL:3 * HL])
        og = jax.nn.sigmoid(gates[:, 3 * HL:4 * HL])
        cs = fg * cs + ig * gg
        hs = og * jnp.tanh(cs)
    hist_out = jnp.dot(hs, Wlo_ref[...], preferred_element_type=f32) + blo_ref[...]

    spatial = jnp.maximum(
        jnp.dot(x_ref[...].reshape(GR, 2), Wsp_ref[...],
                preferred_element_type=f32)
        + bsp_ref[...], 0.0)
    sf = self_ref[...].reshape(GR, 5)
    ds = sf[:, 4:5]
    sx = sf[:, 0:1]
    sy = sf[:, 1:2]
    temp = jnp.sqrt(sx * sx + sy * sy)
    temp_ = jnp.where(temp == 0.0, temp + 0.1, temp)
    predx = (ds * sx / temp_ - sf[:, 2:3]) / TAU
    predy = (ds * sy / temp_ - sf[:, 3:4]) / TAU
    pred_ref[...] = jnp.concatenate([predx, predy], axis=1).reshape(GB, N, 2)

    trow2 = _trow3(beta_ref[...], Wc1t_ref[...], bc1_ref[...])
    part_ref[...] = (jnp.dot(hist_out, Wc1b_ref[...], preferred_element_type=f32)
                     + jnp.dot(spatial, Wc1c_ref[...], preferred_element_type=f32)
                     + trow2).reshape(GB, N, SP // 2)


def _layer_update(h, acc, cnt, Wm2_ref, bm2_ref, Wu1a_ref, Wu1b_ref,
                  bu1_ref, Wu2_ref, bu2_ref):
    f32 = jnp.float32
    agg = (jnp.dot(acc, Wm2_ref[...], preferred_element_type=f32)
           + cnt * bm2_ref[...])
    upd = jnp.maximum(
        jnp.dot(h, Wu1a_ref[...], preferred_element_type=f32)
        + jnp.dot(agg, Wu1b_ref[...], preferred_element_type=f32)
        + bu1_ref[...], 0.0)
    return h + jnp.dot(upd, Wu2_ref[...], preferred_element_type=f32) + bu2_ref[...]


def _b_body(h0_ref, acc0_ref, cnt_ref,
            Wm20_ref, bm20_ref, Wu1a0_ref, Wu1b0_ref, bu10_ref, Wu20_ref,
            bu20_ref, Wm1a1_ref, Wm1b1_ref, bm1r1_ref,
            h1_ref, ajt1_ref, ai1_ref):
    f32 = jnp.float32
    h1 = _layer_update(h0_ref[...].reshape(GR, HID),
                       acc0_ref[...].reshape(GR, HID),
                       cnt_ref[...].reshape(GR, 1), Wm20_ref, bm20_ref,
                       Wu1a0_ref, Wu1b0_ref, bu10_ref, Wu20_ref, bu20_ref)
    h1_ref[...] = h1.reshape(GB, N, HID)
    ajt1_ref[...] = jnp.dot(h1, Wm1b1_ref[...],
                            preferred_element_type=f32).reshape(GB, N, HID)
    ai1_ref[...] = (jnp.dot(h1, Wm1a1_ref[...], preferred_element_type=f32)
                    + bm1r1_ref[...]).reshape(GB, N, HID)


def _c_body(h1_ref, acc1_ref, cnt_ref, part_ref, pred_ref,
            Wm21_ref, bm21_ref, Wu1a1_ref, Wu1b1_ref, bu11_ref, Wu21_ref,
            bu21_ref, Wc1a_ref, Wd_ref, bd_ref, out_ref):
    f32 = jnp.float32
    h2 = _layer_update(h1_ref[...].reshape(GR, HID),
                       acc1_ref[...].reshape(GR, HID),
                       cnt_ref[...].reshape(GR, 1), Wm21_ref, bm21_ref,
                       Wu1a1_ref, Wu1b1_ref, bu11_ref, Wu21_ref, bu21_ref)
    hcat = (jnp.dot(h2, Wc1a_ref[...], preferred_element_type=f32)
            + part_ref[...].reshape(GR, SP // 2))
    out_ref[...] = (jnp.dot(jnp.maximum(hcat, 0.0), Wd_ref[...],
                            preferred_element_type=f32)
                    + bd_ref[...]
                    + pred_ref[...].reshape(GR, 2)).reshape(GB, N, 2)


def _bspec(shape):
    return pl.BlockSpec(shape, lambda b: (b,) + (0,) * (len(shape) - 1))


def _wspec(shape):
    return pl.BlockSpec(shape, lambda b: (0,) * len(shape))


_BETA_SPEC = pl.BlockSpec((GB, 1), lambda b: (b, 0))


def _tc_call(body, in_specs, out_specs, out_shapes, args):
    return pl.pallas_call(
        body, grid=(B // GB,), in_specs=in_specs, out_specs=out_specs,
        out_shape=out_shapes)(*args)


@jax.jit
def _run(beta2, x, ped, maskf, selff, posx, posy, gidx, idxl, maskl, hist2,
         Wef, Wet, bemb,
         Wm1a, Wm1b, bm1r, wrow, Wm2, bm2r, Wu1a, Wu1b, bu1r, Wu2, bu2r,
         Whebd, bhebd, WxT, WhT, bg, Wlo, blo, Wsp, bsp,
         Wc1a, Wc1b, Wc1c, Wc1t, bc1, Wd, bd):
    f32 = jnp.float32
    W = lambda s: _wspec(s)

    h0, ajt0, ai0, cnt = _tc_call(
        _a1_body,
        [_BETA_SPEC, _bspec((GB, N, 6)), _bspec((GB, N, K)),
         W((6, HID)), W((3, HID)), W((1, HID)),
         W((HID, HID)), W((HID, HID)), W((1, HID))],
        [_bspec((GB, N, HID)), _bspec((GB, N, HID)), _bspec((GB, N, HID)),
         _bspec((GB, N, 1))],
        [jax.ShapeDtypeStruct((B, N, HID), f32),
         jax.ShapeDtypeStruct((B, N, HID), f32),
         jax.ShapeDtypeStruct((B, N, HID), f32),
         jax.ShapeDtypeStruct((B, N, 1), f32)],
        [beta2, ped, maskf, Wef, Wet, bemb, Wm1a[0], Wm1b[0], bm1r[0]])

    acc0 = _sc_edges(ajt0.reshape(B * N, HID), ai0, posx, posy,
                     gidx, idxl, maskl, wrow[0])

    part, pred = _tc_call(
        _a2_body,
        [_BETA_SPEC, _bspec((GB, N, 2)), _bspec((GB, N, 5)),
         _bspec((GB, N, T * HD)),
         W((T * HD, T * HE)), W((1, T * HE)),
         W((HE, 4 * HL)), W((HL, 4 * HL)), W((1, 4 * HL)),
         W((HL, HLO)), W((1, HLO)), W((2, SP)), W((1, SP)),
         W((HLO, SP // 2)), W((SP, SP // 2)), W((3, SP // 2)),
         W((1, SP // 2))],
        [_bspec((GB, N, SP // 2)), _bspec((GB, N, 2))],
        [jax.ShapeDtypeStruct((B, N, SP // 2), f32),
         jax.ShapeDtypeStruct((B, N, 2), f32)],
        [beta2, x, selff, hist2, Whebd, bhebd, WxT, WhT, bg, Wlo, blo,
         Wsp, bsp, Wc1b, Wc1c, Wc1t, bc1])

    h1, ajt1, ai1 = _tc_call(
        _b_body,
        [_bspec((GB, N, HID)), _bspec((GB, N, HID)), _bspec((GB, N, 1)),
         W((HID, HID)), W((1, HID)), W((HID, HID)), W((HID, HID)),
         W((1, HID)), W((HID, HID)), W((1, HID)),
         W((HID, HID)), W((HID, HID)), W((1, HID))],
        [_bspec((GB, N, HID)), _bspec((GB, N, HID)), _bspec((GB, N, HID))],
        [jax.ShapeDtypeStruct((B, N, HID), f32),
         jax.ShapeDtypeStruct((B, N, HID), f32),
         jax.ShapeDtypeStruct((B, N, HID), f32)],
        [h0, acc0, cnt, Wm2[0], bm2r[0], Wu1a[0], Wu1b[0], bu1r[0],
         Wu2[0], bu2r[0], Wm1a[1], Wm1b[1], bm1r[1]])

    acc1 = _sc_edges(ajt1.reshape(B * N, HID), ai1, posx, posy,
                     gidx, idxl, maskl, wrow[1])

    out = _tc_call(
        _c_body,
        [_bspec((GB, N, HID)), _bspec((GB, N, HID)), _bspec((GB, N, 1)),
         _bspec((GB, N, SP // 2)), _bspec((GB, N, 2)),
         W((HID, HID)), W((1, HID)), W((HID, HID)), W((HID, HID)),
         W((1, HID)), W((HID, HID)), W((1, HID)),
         W((HID, SP // 2)), W((SP // 2, 2)), W((1, 2))],
        [_bspec((GB, N, 2))],
        [jax.ShapeDtypeStruct((B, N, 2), f32)],
        [h1, acc1, cnt, part, pred, Wm2[1], bm2r[1], Wu1a[1], Wu1b[1],
         bu1r[1], Wu2[1], bu2r[1], Wc1a, Wd, bd])
    return out[0] if isinstance(out, (list, tuple)) else out


def kernel(x, beta, ped_features, neigh_ped_mask, self_features, near_ped_idx,
           hist_feature, nei_list, t, W_emb, b_emb, Wm1, bm1, Wm2, bm2,
           Wu1, bu1, Wu2, bu2, W_sp, b_sp, W_he, b_he, Wih, Whh, bih, bhh,
           W_lo, b_lo, W_c1, b_c1, W_d, b_d):
    f32 = jnp.float32
    beta2 = beta.reshape(B, 1).astype(f32)
    maskf = neigh_ped_mask.astype(f32)
    idx = near_ped_idx.astype(jnp.int32)
    idxl = idx.reshape(B, E)
    gidx = (idx + (jnp.arange(B, dtype=jnp.int32) * N)[:, None, None]
            ).reshape(B, E // 128, 128)
    maskl = maskf.reshape(B, E)
    posx = ped_features[:, :, 0].astype(f32)
    posy = ped_features[:, :, 1].astype(f32)
    hist2 = hist_feature.reshape(B, N, T * HD).astype(f32)

    Wef = W_emb[:6].astype(f32)
    Wet = W_emb[6:9].astype(f32)
    bemb = b_emb.reshape(1, HID).astype(f32)
    Wm1a = Wm1[:, :HID, :].astype(f32)
    Wm1b = Wm1[:, HID:2 * HID, :].astype(f32)
    bm1r = bm1.reshape(L, 1, HID).astype(f32)
    wrow = Wm1[:, 2 * HID, :].astype(f32)
    Wm2c = Wm2.astype(f32)
    bm2r = bm2.reshape(L, 1, HID).astype(f32)
    Wu1a = Wu1[:, :HID, :].astype(f32)
    Wu1b = Wu1[:, HID:, :].astype(f32)
    bu1r = bu1.reshape(L, 1, HID).astype(f32)
    Wu2c = Wu2.astype(f32)
    bu2r = bu2.reshape(L, 1, HID).astype(f32)
    Whebd = jnp.zeros((T * HD, T * HE), f32)
    for tt in range(T):
        Whebd = Whebd.at[tt * HD:(tt + 1) * HD, tt * HE:(tt + 1) * HE].set(
            W_he.astype(f32))
    bhebd = jnp.tile(b_he.astype(f32), T).reshape(1, T * HE)
    WxT = Wih.T.astype(f32)
    WhT = Whh.T.astype(f32)
    bg = (bih + bhh).reshape(1, 4 * HL).astype(f32)
    Wlo = W_lo.astype(f32)
    blo = b_lo.reshape(1, HLO).astype(f32)
    Wsp = W_sp.astype(f32)
    bsp = b_sp.reshape(1, SP).astype(f32)
    Wc1a = W_c1[0:HID].astype(f32)
    Wc1b = W_c1[HID:HID + HLO].astype(f32)
    Wc1c = W_c1[HID + HLO:HID + HLO + SP].astype(f32)
    Wc1t = W_c1[HID + HLO + SP:].astype(f32)
    bc1 = b_c1.reshape(1, SP // 2).astype(f32)
    Wd = W_d.astype(f32)
    bd = b_d.reshape(1, 2).astype(f32)

    return _run(beta2, x.astype(f32), ped_features.astype(f32), maskf,
                self_features.astype(f32), posx, posy, gidx, idxl, maskl,
                hist2, Wef, Wet, bemb,
                Wm1a, Wm1b, bm1r, wrow, Wm2c, bm2r, Wu1a, Wu1b, bu1r,
                Wu2c, bu2r,
                Whebd, bhebd, WxT, WhT, bg, Wlo, blo, Wsp, bsp,
                Wc1a, Wc1b, Wc1c, Wc1t, bc1, Wd, bd)


# speedup vs baseline: 2.6056x; 1.0002x over previous
import jax
import jax.numpy as jnp
from jax import lax
from jax.experimental import pallas as pl
from jax.experimental.pallas import tpu as pltpu
from jax.experimental.pallas import tpu_sc as plsc

B, N, K, T = 64, 128, 32, 8
HID, SP, HD, HE, HL, HLO = 64, 64, 2, 32, 48, 32
L = 2
TAU = 2.0
E = N * K
NC, NS, LN = 2, 16, 16
NW = NC * NS
BPW = B // NW
NB = N // LN
DC = HID // LN


EB = LN * K
NCH = HID // LN


def _sc_edges_body(tabflat, ai, posx, posy, gidx, idxl, maskl, wrow, acc,
                   rows0_v, rows1_v, ai_v, px_v, py_v, gi0_v, gi1_v, jl_v,
                   mk_v, wr_v, acc_v, d2_v, sem0, sem1):
    i32 = jnp.int32
    wid = lax.axis_index("s") * NC + lax.axis_index("c")
    pltpu.sync_copy(wrow, wr_v)
    wr_c = [wr_v[pl.ds(c * LN, LN)] for c in range(NCH)]
    bufs = ((rows0_v, gi0_v, sem0), (rows1_v, gi1_v, sem1))

    def fire(blk, par):
        rows, gi, sem = bufs[par]
        return [pltpu.async_copy(tabflat.at[gi.at[j]],
                                 rows.at[pl.ds(j * 128, 128)], sem)
                for j in range(4)]

    for bb in range(BPW):
        b = wid * BPW + bb
        pltpu.sync_copy(ai.at[b], ai_v)
        pltpu.sync_copy(posx.at[b], px_v)
        pltpu.sync_copy(posy.at[b], py_v)
        pltpu.sync_copy(gidx.at[b, pl.ds(0, 4)], gi0_v)
        fire(0, 0)

        def pair_body(bi, carry_pair):
            for par in range(2):
                blk = bi * 2 + par
                rows, gi_cur, sem = bufs[par]
                nrows, ngi, nsem = bufs[1 - par]
                pltpu.sync_copy(idxl.at[b, pl.ds(blk * EB, EB)], jl_v)
                pltpu.sync_copy(maskl.at[b, pl.ds(blk * EB, EB)], mk_v)

                @pl.when(blk + 1 < NB)
                def _prefetch():
                    pltpu.sync_copy(gidx.at[b, pl.ds((blk + 1) * 4, 4)], ngi)
                    fire(blk + 1, 1 - par)

                def pre_body(g, _):
                    node = blk * LN + g // 2
                    nsplat = jnp.full((LN,), node, i32)
                    jv = jl_v[pl.ds(g * LN, LN)]
                    gx = plsc.load_gather(px_v, [jv])
                    gy = plsc.load_gather(py_v, [jv])
                    pxi = plsc.load_gather(px_v, [nsplat])
                    pyi = plsc.load_gather(py_v, [nsplat])
                    rx = pxi - gx
                    ry = pyi - gy
                    d2_v[pl.ds(g * LN, LN)] = rx * rx + ry * ry
                    return _

                plsc.parallel_loop(0, EB // LN, 1, unroll=4,
                                   carry=jnp.int32(0))(pre_body)
                for j in range(4):
                    pltpu.make_async_copy(
                        tabflat.at[gi_cur.at[j]],
                        rows.at[pl.ds(j * 128, 128)], sem).wait()

                for il in range(LN):
                    node = blk * LN + il
                    ai_c = [ai_v[node, pl.ds(c * LN, LN)] for c in range(NCH)]

                    def k_body(k, carry):
                        e = il * K + k
                        esplat = jnp.full((LN,), e, i32)
                        sd2 = plsc.load_gather(d2_v, [esplat])
                        smv = plsc.load_gather(mk_v, [esplat])
                        out = []
                        for c in range(NCH):
                            row = rows[e, pl.ds(c * LN, LN)]
                            v = jnp.maximum(ai_c[c] + row + sd2 * wr_c[c], 0.0)
                            out.append(carry[c] + smv * v)
                        return tuple(out)

                    acc0 = tuple(jnp.zeros((LN,), jnp.float32)
                                 for _ in range(NCH))
                    accs = plsc.parallel_loop(0, K, 1, unroll=2,
                                              carry=acc0)(k_body)
                    for c in range(NCH):
                        acc_v[node, pl.ds(c * LN, LN)] = accs[c]
            return carry_pair

        lax.fori_loop(0, NB // 2, pair_body, 0)
        pltpu.sync_copy(acc_v, acc.at[b])


def _sc_edges(tabflat, ai, posx, posy, gidx, idxl, maskl, wrow):
    mesh = plsc.VectorSubcoreMesh(core_axis_name="c", subcore_axis_name="s")
    return pl.kernel(
        _sc_edges_body,
        mesh=mesh,
        compiler_params=pltpu.CompilerParams(needs_layout_passes=False,
                                             use_tc_tiling_on_sc=False),
        out_type=jax.ShapeDtypeStruct((B, N, HID), jnp.float32),
        scratch_types=[
            pltpu.VMEM((EB, HID), jnp.float32),
            pltpu.VMEM((EB, HID), jnp.float32),
            pltpu.VMEM((N, HID), jnp.float32),
            pltpu.VMEM((N,), jnp.float32),
            pltpu.VMEM((N,), jnp.float32),
            pltpu.VMEM((4, 128), jnp.int32),
            pltpu.VMEM((4, 128), jnp.int32),
            pltpu.VMEM((EB,), jnp.int32),
            pltpu.VMEM((EB,), jnp.float32),
            pltpu.VMEM((HID,), jnp.float32),
            pltpu.VMEM((N, HID), jnp.float32),
            pltpu.VMEM((EB,), jnp.float32),
            pltpu.SemaphoreType.DMA,
            pltpu.SemaphoreType.DMA,
        ],
    )(tabflat, ai, posx, posy, gidx, idxl, maskl, wrow)


GB = 8
GR = GB * N


def _trow3(beta_v, W3, brow):
    t = (beta_v * W3[0:1, :] + jnp.sin(beta_v) * W3[1:2, :]
         + jnp.cos(beta_v) * W3[2:3, :] + brow)
    H = t.shape[1]
    return jnp.broadcast_to(t.reshape(GB, 1, H), (GB, N, H)).reshape(GR, H)


def _a1_body(beta_ref, ped_ref, maskf_ref,
             Wef_ref, Wet_ref, bemb_ref, Wm1a0_ref, Wm1b0_ref, bm1r0_ref,
             h0_ref, ajt0_ref, ai0_ref, cnt_ref):
    f32 = jnp.float32
    trow = _trow3(beta_ref[...], Wet_ref[...], bemb_ref[...])
    ped = ped_ref[...].reshape(GR, 6)
    h = jnp.dot(ped, Wef_ref[...], preferred_element_type=f32) + trow
    h0_ref[...] = h.reshape(GB, N, HID)
    ajt0_ref[...] = jnp.dot(h, Wm1b0_ref[...],
                            preferred_element_type=f32).reshape(GB, N, HID)
    ai0_ref[...] = (jnp.dot(h, Wm1a0_ref[...], preferred_element_type=f32)
                    + bm1r0_ref[...]).reshape(GB, N, HID)
    cnt_ref[...] = jnp.sum(maskf_ref[...], axis=2, keepdims=True)


def _a2_body(beta_ref, x_ref, self_ref, hist_ref,
             Whebd_ref, bhebd_ref, WxT_ref, WhT_ref, bg_ref, Wlo_ref, blo_ref,
             Wsp_ref, bsp_ref, Wc1b_ref, Wc1c_ref, Wc1t_ref, bc1_ref,
             part_ref, pred_ref):
    f32 = jnp.float32
    he = jnp.maximum(
        jnp.dot(hist_ref[...].reshape(GR, T * HD), Whebd_ref[...],
                preferred_element_type=f32)
        + bhebd_ref[...], 0.0)
    hs = jnp.zeros((GR, HL), f32)
    cs = jnp.zeros((GR, HL), f32)
    for tt in range(T):
        xt = he[:, HE * tt:HE * (tt + 1)]
        gates = (jnp.dot(xt, WxT_ref[...], preferred_element_type=f32)
                 + jnp.dot(hs, WhT_ref[...], preferred_element_type=f32)
                 + bg_ref[...])
        ig = jax.nn.sigmoid(gates[:, 0:HL])
        fg = jax.nn.sigmoid(gates[:, HL:2 * HL])
        gg = jnp.tanh(gates[:, 2 * HL:3 * HL])
        og = jax.nn.sigmoid(gates[:, 3 * HL:4 * HL])
        cs = fg * cs + ig * gg
        hs = og * jnp.tanh(cs)
    hist_out = jnp.dot(hs, Wlo_ref[...], preferred_element_type=f32) + blo_ref[...]

    spatial = jnp.maximum(
        jnp.dot(x_ref[...].reshape(GR, 2), Wsp_ref[...],
                preferred_element_type=f32)
        + bsp_ref[...], 0.0)
    sf = self_ref[...].reshape(GR, 5)
    ds = sf[:, 4:5]
    sx = sf[:, 0:1]
    sy = sf[:, 1:2]
    temp = jnp.sqrt(sx * sx + sy * sy)
    temp_ = jnp.where(temp == 0.0, temp + 0.1, temp)
    predx = (ds * sx / temp_ - sf[:, 2:3]) / TAU
    predy = (ds * sy / temp_ - sf[:, 3:4]) / TAU
    pred_ref[...] = jnp.concatenate([predx, predy], axis=1).reshape(GB, N, 2)

    trow2 = _trow3(beta_ref[...], Wc1t_ref[...], bc1_ref[...])
    part_ref[...] = (jnp.dot(hist_out, Wc1b_ref[...], preferred_element_type=f32)
                     + jnp.dot(spatial, Wc1c_ref[...], preferred_element_type=f32)
                     + trow2).reshape(GB, N, SP // 2)


def _layer_update(h, acc, cnt, Wm2_ref, bm2_ref, Wu1a_ref, Wu1b_ref,
                  bu1_ref, Wu2_ref, bu2_ref):
    f32 = jnp.float32
    agg = (jnp.dot(acc, Wm2_ref[...], preferred_element_type=f32)
           + cnt * bm2_ref[...])
    upd = jnp.maximum(
        jnp.dot(h, Wu1a_ref[...], preferred_element_type=f32)
        + jnp.dot(agg, Wu1b_ref[...], preferred_element_type=f32)
        + bu1_ref[...], 0.0)
    return h + jnp.dot(upd, Wu2_ref[...], preferred_element_type=f32) + bu2_ref[...]


def _b_body(h0_ref, acc0_ref, cnt_ref,
            Wm20_ref, bm20_ref, Wu1a0_ref, Wu1b0_ref, bu10_ref, Wu20_ref,
            bu20_ref, Wm1a1_ref, Wm1b1_ref, bm1r1_ref,
            h1_ref, ajt1_ref, ai1_ref):
    f32 = jnp.float32
    h1 = _layer_update(h0_ref[...].reshape(GR, HID),
                       acc0_ref[...].reshape(GR, HID),
                       cnt_ref[...].reshape(GR, 1), Wm20_ref, bm20_ref,
                       Wu1a0_ref, Wu1b0_ref, bu10_ref, Wu20_ref, bu20_ref)
    h1_ref[...] = h1.reshape(GB, N, HID)
    ajt1_ref[...] = jnp.dot(h1, Wm1b1_ref[...],
                            preferred_element_type=f32).reshape(GB, N, HID)
    ai1_ref[...] = (jnp.dot(h1, Wm1a1_ref[...], preferred_element_type=f32)
                    + bm1r1_ref[...]).reshape(GB, N, HID)


def _c_body(h1_ref, acc1_ref, cnt_ref, part_ref, pred_ref,
            Wm21_ref, bm21_ref, Wu1a1_ref, Wu1b1_ref, bu11_ref, Wu21_ref,
            bu21_ref, Wc1a_ref, Wd_ref, bd_ref, out_ref):
    f32 = jnp.float32
    h2 = _layer_update(h1_ref[...].reshape(GR, HID),
                       acc1_ref[...].reshape(GR, HID),
                       cnt_ref[...].reshape(GR, 1), Wm21_ref, bm21_ref,
                       Wu1a1_ref, Wu1b1_ref, bu11_ref, Wu21_ref, bu21_ref)
    hcat = (jnp.dot(h2, Wc1a_ref[...], preferred_element_type=f32)
            + part_ref[...].reshape(GR, SP // 2))
    out_ref[...] = (jnp.dot(jnp.maximum(hcat, 0.0), Wd_ref[...],
                            preferred_element_type=f32)
                    + bd_ref[...]
                    + pred_ref[...].reshape(GR, 2)).reshape(GB, N, 2)


def _bspec(shape):
    return pl.BlockSpec(shape, lambda b: (b,) + (0,) * (len(shape) - 1))


def _wspec(shape):
    return pl.BlockSpec(shape, lambda b: (0,) * len(shape))


_BETA_SPEC = pl.BlockSpec((GB, 1), lambda b: (b, 0))


def _tc_call(body, in_specs, out_specs, out_shapes, args):
    return pl.pallas_call(
        body, grid=(B // GB,), in_specs=in_specs, out_specs=out_specs,
        out_shape=out_shapes)(*args)


@jax.jit
def _run(beta2, x, ped, maskf, selff, posx, posy, gidx, idxl, maskl, hist2,
         Wef, Wet, bemb,
         Wm1a, Wm1b, bm1r, wrow, Wm2, bm2r, Wu1a, Wu1b, bu1r, Wu2, bu2r,
         Whebd, bhebd, WxT, WhT, bg, Wlo, blo, Wsp, bsp,
         Wc1a, Wc1b, Wc1c, Wc1t, bc1, Wd, bd):
    f32 = jnp.float32
    W = lambda s: _wspec(s)

    h0, ajt0, ai0, cnt = _tc_call(
        _a1_body,
        [_BETA_SPEC, _bspec((GB, N, 6)), _bspec((GB, N, K)),
         W((6, HID)), W((3, HID)), W((1, HID)),
         W((HID, HID)), W((HID, HID)), W((1, HID))],
        [_bspec((GB, N, HID)), _bspec((GB, N, HID)), _bspec((GB, N, HID)),
         _bspec((GB, N, 1))],
        [jax.ShapeDtypeStruct((B, N, HID), f32),
         jax.ShapeDtypeStruct((B, N, HID), f32),
         jax.ShapeDtypeStruct((B, N, HID), f32),
         jax.ShapeDtypeStruct((B, N, 1), f32)],
        [beta2, ped, maskf, Wef, Wet, bemb, Wm1a[0], Wm1b[0], bm1r[0]])

    acc0 = _sc_edges(ajt0.reshape(B * N, HID), ai0, posx, posy,
                     gidx, idxl, maskl, wrow[0])

    part, pred = _tc_call(
        _a2_body,
        [_BETA_SPEC, _bspec((GB, N, 2)), _bspec((GB, N, 5)),
         _bspec((GB, N, T * HD)),
         W((T * HD, T * HE)), W((1, T * HE)),
         W((HE, 4 * HL)), W((HL, 4 * HL)), W((1, 4 * HL)),
         W((HL, HLO)), W((1, HLO)), W((2, SP)), W((1, SP)),
         W((HLO, SP // 2)), W((SP, SP // 2)), W((3, SP // 2)),
         W((1, SP // 2))],
        [_bspec((GB, N, SP // 2)), _bspec((GB, N, 2))],
        [jax.ShapeDtypeStruct((B, N, SP // 2), f32),
         jax.ShapeDtypeStruct((B, N, 2), f32)],
        [beta2, x, selff, hist2, Whebd, bhebd, WxT, WhT, bg, Wlo, blo,
         Wsp, bsp, Wc1b, Wc1c, Wc1t, bc1])

    h1, ajt1, ai1 = _tc_call(
        _b_body,
        [_bspec((GB, N, HID)), _bspec((GB, N, HID)), _bspec((GB, N, 1)),
         W((HID, HID)), W((1, HID)), W((HID, HID)), W((HID, HID)),
         W((1, HID)), W((HID, HID)), W((1, HID)),
         W((HID, HID)), W((HID, HID)), W((1, HID))],
        [_bspec((GB, N, HID)), _bspec((GB, N, HID)), _bspec((GB, N, HID))],
        [jax.ShapeDtypeStruct((B, N, HID), f32),
         jax.ShapeDtypeStruct((B, N, HID), f32),
         jax.ShapeDtypeStruct((B, N, HID), f32)],
        [h0, acc0, cnt, Wm2[0], bm2r[0], Wu1a[0], Wu1b[0], bu1r[0],
         Wu2[0], bu2r[0], Wm1a[1], Wm1b[1], bm1r[1]])

    acc1 = _sc_edges(ajt1.reshape(B * N, HID), ai1, posx, posy,
                     gidx, idxl, maskl, wrow[1])

    out = _tc_call(
        _c_body,
        [_bspec((GB, N, HID)), _bspec((GB, N, HID)), _bspec((GB, N, 1)),
         _bspec((GB, N, SP // 2)), _bspec((GB, N, 2)),
         W((HID, HID)), W((1, HID)), W((HID, HID)), W((HID, HID)),
         W((1, HID)), W((HID, HID)), W((1, HID)),
         W((HID, SP // 2)), W((SP // 2, 2)), W((1, 2))],
        [_bspec((GB, N, 2))],
        [jax.ShapeDtypeStruct((B, N, 2), f32)],
        [h1, acc1, cnt, part, pred, Wm2[1], bm2r[1], Wu1a[1], Wu1b[1],
         bu1r[1], Wu2[1], bu2r[1], Wc1a, Wd, bd])
    return out[0] if isinstance(out, (list, tuple)) else out


def kernel(x, beta, ped_features, neigh_ped_mask, self_features, near_ped_idx,
           hist_feature, nei_list, t, W_emb, b_emb, Wm1, bm1, Wm2, bm2,
           Wu1, bu1, Wu2, bu2, W_sp, b_sp, W_he, b_he, Wih, Whh, bih, bhh,
           W_lo, b_lo, W_c1, b_c1, W_d, b_d):
    f32 = jnp.float32
    beta2 = beta.reshape(B, 1).astype(f32)
    maskf = neigh_ped_mask.astype(f32)
    idx = near_ped_idx.astype(jnp.int32)
    idxl = idx.reshape(B, E)
    gidx = (idx + (jnp.arange(B, dtype=jnp.int32) * N)[:, None, None]
            ).reshape(B, E // 128, 128)
    maskl = maskf.reshape(B, E)
    posx = ped_features[:, :, 0].astype(f32)
    posy = ped_features[:, :, 1].astype(f32)
    hist2 = hist_feature.reshape(B, N, T * HD).astype(f32)

    Wef = W_emb[:6].astype(f32)
    Wet = W_emb[6:9].astype(f32)
    bemb = b_emb.reshape(1, HID).astype(f32)
    Wm1a = Wm1[:, :HID, :].astype(f32)
    Wm1b = Wm1[:, HID:2 * HID, :].astype(f32)
    bm1r = bm1.reshape(L, 1, HID).astype(f32)
    wrow = Wm1[:, 2 * HID, :].astype(f32)
    Wm2c = Wm2.astype(f32)
    bm2r = bm2.reshape(L, 1, HID).astype(f32)
    Wu1a = Wu1[:, :HID, :].astype(f32)
    Wu1b = Wu1[:, HID:, :].astype(f32)
    bu1r = bu1.reshape(L, 1, HID).astype(f32)
    Wu2c = Wu2.astype(f32)
    bu2r = bu2.reshape(L, 1, HID).astype(f32)
    Whebd = jnp.zeros((T * HD, T * HE), f32)
    for tt in range(T):
        Whebd = Whebd.at[tt * HD:(tt + 1) * HD, tt * HE:(tt + 1) * HE].set(
            W_he.astype(f32))
    bhebd = jnp.tile(b_he.astype(f32), T).reshape(1, T * HE)
    WxT = Wih.T.astype(f32)
    WhT = Whh.T.astype(f32)
    bg = (bih + bhh).reshape(1, 4 * HL).astype(f32)
    Wlo = W_lo.astype(f32)
    blo = b_lo.reshape(1, HLO).astype(f32)
    Wsp = W_sp.astype(f32)
    bsp = b_sp.reshape(1, SP).astype(f32)
    Wc1a = W_c1[0:HID].astype(f32)
    Wc1b = W_c1[HID:HID + HLO].astype(f32)
    Wc1c = W_c1[HID + HLO:HID + HLO + SP].astype(f32)
    Wc1t = W_c1[HID + HLO + SP:].astype(f32)
    bc1 = b_c1.reshape(1, SP // 2).astype(f32)
    Wd = W_d.astype(f32)
    bd = b_d.reshape(1, 2).astype(f32)

    return _run(beta2, x.astype(f32), ped_features.astype(f32), maskf,
                self_features.astype(f32), posx, posy, gidx, idxl, maskl,
                hist2, Wef, Wet, bemb,
                Wm1a, Wm1b, bm1r, wrow, Wm2c, bm2r, Wu1a, Wu1b, bu1r,
                Wu2c, bu2r,
                Whebd, bhebd, WxT, WhT, bg, Wlo, blo, Wsp, bsp,
                Wc1a, Wc1b, Wc1c, Wc1t, bc1, Wd, bd)
